# baseline (speedup 1.0000x reference)
import functools

import jax
import jax.numpy as jnp
from jax import lax
from jax.experimental import pallas as pl
from jax.experimental.pallas import tpu as pltpu
from jax.experimental.pallas import tpu_sc as plsc

N = 10000
E = 320000
D = 128
DE = 4
L = 4
OUT = 21

NC = 2
NS = 16
NW = NC * NS

BLK = 128
CW = 79
EW = CW * BLK
EPAD = NW * EW

SINK = 112
NACC = N + SINK
RPT = NACC // NS
ZR = 16

SBLK = 64
SCW = 79
SEW = SCW * SBLK

BN = 2000
BE = 2048

_MESH = dict(core_axis_name="c", subcore_axis_name="s", num_cores=NC,
             num_subcores=NS)


def _wid():
    return lax.axis_index("s") * NC + lax.axis_index("c")


@functools.partial(
    pl.kernel,
    out_type=jax.ShapeDtypeStruct((NW, CW, BLK), jnp.float32),
    mesh=plsc.VectorSubcoreMesh(**_MESH),
    scratch_types=[
        pltpu.VMEM((CW, BLK), jnp.int32),
        pltpu.VMEM((CW, BLK), jnp.int32),
        pltpu.VMEM((6, BLK), jnp.float32),
        pltpu.VMEM((CW, BLK), jnp.float32),
        pltpu.SemaphoreType.DMA,
    ],
)
def _radial_sc(x0_h, x1_h, x2_h, idxr_h, idxc_h, out_h, ir_v, ic_v, buf,
               o_v, sem):
    wid = _wid()
    pltpu.sync_copy(idxr_h.at[wid], ir_v)
    pltpu.sync_copy(idxc_h.at[wid], ic_v)

    def chunk(j, carry):
        pltpu.async_copy(x0_h.at[ir_v.at[j]], buf.at[0], sem)
        pltpu.async_copy(x1_h.at[ir_v.at[j]], buf.at[1], sem)
        pltpu.async_copy(x2_h.at[ir_v.at[j]], buf.at[2], sem)
        pltpu.async_copy(x0_h.at[ic_v.at[j]], buf.at[3], sem)
        pltpu.async_copy(x1_h.at[ic_v.at[j]], buf.at[4], sem)
        cp = pltpu.async_copy(x2_h.at[ic_v.at[j]], buf.at[5], sem)
        for k in range(5):
            pltpu.make_async_copy(x0_h.at[ir_v.at[j]], buf.at[k], sem).wait()
        cp.wait()
        for g in range(BLK // 16):
            sl = pl.ds(g * 16, 16)
            d0 = buf[0, sl] - buf[3, sl]
            d1 = buf[1, sl] - buf[4, sl]
            d2 = buf[2, sl] - buf[5, sl]
            o_v[j, sl] = d0 * d0 + d1 * d1 + d2 * d2
        return carry

    lax.fori_loop(0, CW, chunk, 0)
    pltpu.sync_copy(o_v, out_h.at[wid])


EH = EPAD // 2
GBLK = 64
GC = 79
GN = 2
TCW = GN * GC
TEW = TCW * GBLK
TROWS = 632


@functools.partial(
    pl.kernel,
    out_type=jax.ShapeDtypeStruct((NC, EH, D), jnp.float32),
    mesh=plsc.VectorSubcoreMesh(**_MESH),
    scratch_types=[
        pltpu.VMEM((2 * GC, GBLK), jnp.int32),
        pltpu.VMEM((3, GBLK, D), jnp.float32),
        pltpu.VMEM_SHARED((N, D), jnp.float32),
        pltpu.SemaphoreType.DMA,
        pltpu.SemaphoreType.DMA,
        pltpu.SemaphoreType.DMA,
    ],
)
def _gather_sc(hp_h, hq_h, idx_h, out_h, idx_v, buf, tab, sem_g, sem_o,
               sem_i):
    cid = lax.axis_index("c")
    sid = lax.axis_index("s")

    r0 = sid * TROWS
    pltpu.sync_copy(idx_h.at[cid, sid, 0], idx_v.at[pl.ds(0, GC)])

    @pl.when(jnp.logical_and(cid == 0, sid < NS - 1))
    def _stage_p():
        pltpu.sync_copy(hp_h.at[pl.ds(r0, TROWS)], tab.at[pl.ds(r0, TROWS)])

    @pl.when(jnp.logical_and(cid == 0, sid == NS - 1))
    def _stage_p_last():
        pltpu.sync_copy(hp_h.at[pl.ds((NS - 1) * TROWS, N - (NS - 1) * TROWS)],
                        tab.at[pl.ds((NS - 1) * TROWS, N - (NS - 1) * TROWS)])

    @pl.when(jnp.logical_and(cid == 1, sid < NS - 1))
    def _stage_q():
        pltpu.sync_copy(hq_h.at[pl.ds(r0, TROWS)], tab.at[pl.ds(r0, TROWS)])

    @pl.when(jnp.logical_and(cid == 1, sid == NS - 1))
    def _stage_q_last():
        pltpu.sync_copy(hq_h.at[pl.ds((NS - 1) * TROWS, N - (NS - 1) * TROWS)],
                        tab.at[pl.ds((NS - 1) * TROWS, N - (NS - 1) * TROWS)])

    plsc.subcore_barrier()

    base = sid * TEW
    pltpu.async_copy(idx_h.at[cid, sid, 1], idx_v.at[pl.ds(GC, GC)], sem_i)
    pltpu.async_copy(tab.at[idx_v.at[0]], buf.at[0], sem_g)
    pltpu.async_copy(tab.at[idx_v.at[1]], buf.at[1], sem_g)

    def _irow(j):
        g = lax.div(j, GC)
        return lax.rem(g, 2) * GC + lax.rem(j, GC)

    def chunk(j, carry):
        slot = lax.rem(j, 3)

        @pl.when(jnp.logical_and(lax.rem(j, GC) == GC - 3,
                                 lax.div(j, GC) < GN - 1))
        def _wait_idx():
            g = lax.div(j, GC) + 1
            pltpu.make_async_copy(
                idx_h.at[cid, sid, g],
                idx_v.at[pl.ds(lax.rem(g, 2) * GC, GC)], sem_i).wait()

        pltpu.make_async_copy(tab.at[idx_v.at[_irow(j)]], buf.at[slot],
                              sem_g).wait()

        @pl.when(jnp.logical_and(lax.rem(j, GC) == GC - 1,
                                 lax.div(j, GC) < GN - 2))
        def _load_idx():
            g = lax.div(j, GC) + 2
            pltpu.async_copy(idx_h.at[cid, sid, g],
                             idx_v.at[pl.ds(lax.rem(g, 2) * GC, GC)], sem_i)

        @pl.when(j >= 1)
        def _drain_prev_out():
            pltpu.make_async_copy(
                buf.at[lax.rem(j + 2, 3)],
                out_h.at[cid].at[pl.ds(base + (j - 1) * GBLK, GBLK)],
                sem_o).wait()

        pltpu.async_copy(buf.at[slot],
                         out_h.at[cid].at[pl.ds(base + j * GBLK, GBLK)],
                         sem_o)

        @pl.when(j + 2 < TCW)
        def _prefetch():
            pltpu.async_copy(tab.at[idx_v.at[_irow(j + 2)]],
                             buf.at[lax.rem(j + 2, 3)], sem_g)
        return carry

    lax.fori_loop(0, TCW, chunk, 0)
    pltpu.make_async_copy(
        buf.at[lax.rem(TCW - 1, 3)],
        out_h.at[cid].at[pl.ds(base + (TCW - 1) * GBLK, GBLK)],
        sem_o).wait()


@functools.partial(
    pl.kernel,
    out_type=jax.ShapeDtypeStruct((NC, NACC, D), jnp.float32),
    mesh=plsc.VectorSubcoreMesh(**_MESH),
    scratch_types=[
        pltpu.VMEM((SCW, SBLK), jnp.int32),
        pltpu.VMEM((2, SBLK, D), jnp.float32),
        pltpu.VMEM((ZR, D), jnp.float32),
        pltpu.VMEM_SHARED((NACC, D), jnp.float32),
        pltpu.SemaphoreType.DMA,
    ],
)
def _scatter_sc(ef2_h, idxs_h, out_h, idx_v, upd, zbuf, acc, sem):
    cid = lax.axis_index("c")
    sid = lax.axis_index("s")
    wid = sid * NC + cid

    def zrow(r, carry):
        for g in range(D // 16):
            zbuf[r, pl.ds(g * 16, 16)] = jnp.zeros((16,), jnp.float32)
        return carry

    lax.fori_loop(0, ZR, zrow, 0)

    base_r = sid * RPT
    for t in range(RPT // ZR):
        pltpu.sync_copy(zbuf, acc.at[pl.ds(base_r + t * ZR, ZR)])
    rem = RPT - (RPT // ZR) * ZR
    if rem:
        pltpu.sync_copy(zbuf.at[pl.ds(0, rem)],
                        acc.at[pl.ds(base_r + (RPT // ZR) * ZR, rem)])
    plsc.subcore_barrier()

    pltpu.sync_copy(idxs_h.at[wid], idx_v)
    base_e = wid * SEW
    pltpu.async_copy(ef2_h.at[pl.ds(base_e, SBLK)], upd.at[0], sem)

    def chunk(j, carry):
        slot = lax.rem(j, 2)
        nslot = lax.rem(j + 1, 2)

        @pl.when(j + 1 < SCW)
        def _prefetch():
            pltpu.async_copy(ef2_h.at[pl.ds(base_e + (j + 1) * SBLK, SBLK)],
                             upd.at[nslot], sem)

        pltpu.make_async_copy(ef2_h.at[pl.ds(base_e + j * SBLK, SBLK)],
                              upd.at[slot], sem).wait()
        pltpu.sync_copy(upd.at[slot], acc.at[idx_v.at[j]], add=True)
        return carry

    lax.fori_loop(0, SCW, chunk, 0)
    plsc.subcore_barrier()
    pltpu.sync_copy(acc.at[pl.ds(sid * RPT, RPT)],
                    out_h.at[cid].at[pl.ds(sid * RPT, RPT)])


def _silu(t):
    return t * jax.nn.sigmoid(t)


def _embed_body(h0_r, we_r, be_r, wa_r, wb_r, h_r, hp_r, hq_r):
    h = jnp.dot(h0_r[...], we_r[...],
                preferred_element_type=jnp.float32) + be_r[...]
    h_r[...] = h
    hp_r[...] = jnp.dot(h, wa_r[...], preferred_element_type=jnp.float32)
    hq_r[...] = jnp.dot(h, wb_r[...], preferred_element_type=jnp.float32)


def _edge_body(pre_r, ea8_r, weg_r, b1_r, w2_r, b2_r, out_r):
    f32 = jnp.float32
    t = pre_r[0] + pre_r[1] + jnp.dot(ea8_r[...], weg_r[...],
                                      preferred_element_type=f32) + b1_r[...]
    t = _silu(t)
    u = jnp.dot(t.astype(jnp.bfloat16), w2_r[...],
                preferred_element_type=f32) + b2_r[...]
    out_r[...] = _silu(u)


def _node_core(h, agga_r, aggb_r, h0_r, a_r, b_r, c_r, b1_r, w2_r, b2_r):
    f32 = jnp.float32
    agg = (agga_r[0] + agga_r[1]) + (aggb_r[0] + aggb_r[1])
    t = (jnp.dot(h, a_r[...], preferred_element_type=f32)
         + jnp.dot(agg, b_r[...], preferred_element_type=f32)
         + jnp.dot(h0_r[...], c_r[...], preferred_element_type=f32)
         + b1_r[...])
    m = jnp.dot(_silu(t), w2_r[...], preferred_element_type=f32) + b2_r[...]
    return h + m


def _node_body(h_r, agga_r, aggb_r, h0_r, a_r, b_r, c_r, b1_r, w2_r, b2_r,
               wa_r, wb_r, hn_r, hp_r, hq_r):
    hn = _node_core(h_r[...], agga_r, aggb_r, h0_r, a_r, b_r, c_r, b1_r,
                    w2_r, b2_r)
    hn_r[...] = hn
    hp_r[...] = jnp.dot(hn, wa_r[...], preferred_element_type=jnp.float32)
    hq_r[...] = jnp.dot(hn, wb_r[...], preferred_element_type=jnp.float32)


def _final_body(h_r, agga_r, aggb_r, h0_r, a_r, b_r, c_r, b1_r, w2_r, b2_r,
                ndw1_r, ndb1_r, ndw2_r, ndb2_r,
                gdw1_r, gdb1_r, gdw2_r, gdb2_r, out_r):
    f32 = jnp.float32
    hn = _node_core(h_r[...], agga_r, aggb_r, h0_r, a_r, b_r, c_r, b1_r,
                    w2_r, b2_r)
    t = jnp.dot(_silu(jnp.dot(hn, ndw1_r[...], preferred_element_type=f32)
                      + ndb1_r[...]),
                ndw2_r[...], preferred_element_type=f32) + ndb2_r[...]
    u = _silu(jnp.dot(t, gdw1_r[...], preferred_element_type=f32)
              + gdb1_r[...])
    out_r[...] = jnp.dot(u, gdw2_r[...],
                         preferred_element_type=f32) + gdb2_r[...]


def _full(shape):
    return pl.BlockSpec(shape, lambda i: tuple(0 for _ in shape))


def _nblk():
    return pl.BlockSpec((BN, D), lambda i: (i, 0))


def _aggblk():
    return pl.BlockSpec((NC, BN, D), lambda i: (0, i, 0))


_W = _full((D, D))
_B = _full((1, D))


def _embed_call(h0, we, be, wa, wb):
    return pl.pallas_call(
        _embed_body,
        grid=(N // BN,),
        in_specs=[_nblk(), _W, _B, _W, _W],
        out_specs=[_nblk(), _nblk(), _nblk()],
        out_shape=[jax.ShapeDtypeStruct((N, D), jnp.float32)] * 3,
    )(h0, we, be, wa, wb)


def _edge_call(pre2, ea8h, weg, b1, w2, b2):
    eblk = pl.BlockSpec((BE, D), lambda i: (i, 0))
    return pl.pallas_call(
        _edge_body,
        grid=(EH // BE,),
        in_specs=[pl.BlockSpec((NC, BE, D), lambda i: (0, i, 0)),
                  pl.BlockSpec((BE, 8), lambda i: (i, 0)),
                  _full((8, D)), _B, _W, _B],
        out_specs=eblk,
        out_shape=jax.ShapeDtypeStruct((EH, D), jnp.float32),
    )(pre2, ea8h, weg, b1, w2.astype(jnp.bfloat16), b2)


def _node_call(h, agga, aggb, h0, a, b, c, b1, w2, b2, wa, wb):
    return pl.pallas_call(
        _node_body,
        grid=(N // BN,),
        in_specs=[_nblk(), _aggblk(), _aggblk(), _nblk(), _W, _W, _W, _B,
                  _W, _B, _W, _W],
        out_specs=[_nblk(), _nblk(), _nblk()],
        out_shape=[jax.ShapeDtypeStruct((N, D), jnp.float32)] * 3,
    )(h, agga, aggb, h0, a, b, c, b1, w2, b2, wa, wb)


def _final_call(h, agga, aggb, h0, a, b, c, b1, w2, b2, ndw1, ndb1, ndw2,
                ndb2, gdw1, gdb1, gdw2p, gdb2p):
    return pl.pallas_call(
        _final_body,
        grid=(N // BN,),
        in_specs=[_nblk(), _aggblk(), _aggblk(), _nblk(), _W, _W, _W, _B,
                  _W, _B, _W, _B, _W, _B, _W, _B, _W, _B],
        out_specs=_nblk(),
        out_shape=jax.ShapeDtypeStruct((N, D), jnp.float32),
    )(h, agga, aggb, h0, a, b, c, b1, w2, b2, ndw1, ndb1, ndw2, ndb2,
      gdw1, gdb1, gdw2p, gdb2p)


def kernel(h0, x, edges, edge_attr, emb_W, emb_b, e_W1, e_b1, e_W2, e_b2,
           n_W1, n_b1, n_W2, n_b2, nd_W1, nd_b1, nd_W2, nd_b2,
           gd_W1, gd_b1, gd_W2, gd_b2):
    f32 = jnp.float32
    row = edges[0]
    col = edges[1]
    pad = EPAD - E
    pidx = jnp.arange(pad, dtype=jnp.int32)
    row_gf = jnp.concatenate([row, pidx % N])
    col_gf = jnp.concatenate([col, (pidx * 7 + 3) % N])
    row_g = row_gf.reshape(NW, CW, BLK)
    col_g = col_gf.reshape(NW, CW, BLK)
    row_sf = jnp.concatenate([row, N + (pidx % SINK)])
    idx5 = [jnp.stack([row_gf[h * EH:(h + 1) * EH],
                       col_gf[h * EH:(h + 1) * EH]]
                      ).reshape(2, NS, GN, GC, GBLK) for h in range(2)]
    row_s = [row_sf[h * EH:(h + 1) * EH].reshape(NW, SCW, SBLK)
             for h in range(2)]

    ea_pad = jnp.pad(edge_attr, ((0, pad), (0, 0)))
    x0, x1, x2 = x[:, 0], x[:, 1], x[:, 2]

    wa = e_W1[:, :D, :]
    wb = e_W1[:, D:2 * D, :]
    wg = e_W1[:, 2 * D:, :]
    na = n_W1[:, :D, :]
    nb = n_W1[:, D:2 * D, :]
    nc_ = n_W1[:, 2 * D:, :]
    gdw2p = jnp.pad(gd_W2, ((0, 0), (0, D - OUT)))
    gdb2p = jnp.pad(gd_b2, (0, D - OUT))[None, :]

    h, hp, hq = _embed_call(h0, emb_W, emb_b[None, :], wa[0], wb[0])
    radial = _radial_sc(x0, x1, x2, row_g, col_g)
    ea8f = jnp.concatenate(
        [radial.reshape(EPAD, 1), ea_pad, jnp.zeros((EPAD, 3), f32)], axis=1)
    ea8 = [ea8f[h * EH:(h + 1) * EH] for h in range(2)]

    pred = None
    for i in range(L):
        weg = jnp.concatenate([wg[i], jnp.zeros((3, D), f32)], axis=0)
        b1 = e_b1[i][None, :]
        b2 = e_b2[i][None, :]
        pre_a = _gather_sc(hp, hq, idx5[0])
        ef2_a = _edge_call(pre_a, ea8[0], weg, b1, e_W2[i], b2)
        pre_b = _gather_sc(hp, hq, idx5[1])
        ef2_b = _edge_call(pre_b, ea8[1], weg, b1, e_W2[i], b2)
        agg_a = _scatter_sc(ef2_a, row_s[0])
        agg_b = _scatter_sc(ef2_b, row_s[1])
        if i < L - 1:
            h, hp, hq = _node_call(h, agg_a, agg_b, h0, na[i], nb[i],
                                   nc_[i], n_b1[i][None, :], n_W2[i],
                                   n_b2[i][None, :], wa[i + 1], wb[i + 1])
        else:
            pred = _final_call(h, agg_a, agg_b, h0, na[i], nb[i], nc_[i],
                               n_b1[i][None, :], n_W2[i], n_b2[i][None, :],
                               nd_W1, nd_b1[None, :], nd_W2, nd_b2[None, :],
                               gd_W1, gd_b1[None, :], gdw2p, gdb2p)
    return pred[:, :OUT]


# speedup vs baseline: 1.0381x; 1.0381x over previous
import functools

import jax
import jax.numpy as jnp
from jax import lax
from jax.experimental import pallas as pl
from jax.experimental.pallas import tpu as pltpu
from jax.experimental.pallas import tpu_sc as plsc

N = 10000
E = 320000
D = 128
DE = 4
L = 4
OUT = 21

NC = 2
NS = 16
NW = NC * NS

BLK = 128
CW = 79
EW = CW * BLK
EPAD = NW * EW

SINK = 112
NACC = N + SINK
RPT = NACC // NS
ZR = 16

SBLK = 128
SCW = 79
SEW = SCW * SBLK

BN = 2000
BE = 4096

_MESH = dict(core_axis_name="c", subcore_axis_name="s", num_cores=NC,
             num_subcores=NS)


def _wid():
    return lax.axis_index("s") * NC + lax.axis_index("c")


@functools.partial(
    pl.kernel,
    out_type=jax.ShapeDtypeStruct((NW, CW, BLK), jnp.float32),
    mesh=plsc.VectorSubcoreMesh(**_MESH),
    scratch_types=[
        pltpu.VMEM((CW, BLK), jnp.int32),
        pltpu.VMEM((CW, BLK), jnp.int32),
        pltpu.VMEM((6, BLK), jnp.float32),
        pltpu.VMEM((CW, BLK), jnp.float32),
        pltpu.SemaphoreType.DMA,
    ],
)
def _radial_sc(x0_h, x1_h, x2_h, idxr_h, idxc_h, out_h, ir_v, ic_v, buf,
               o_v, sem):
    wid = _wid()
    pltpu.sync_copy(idxr_h.at[wid], ir_v)
    pltpu.sync_copy(idxc_h.at[wid], ic_v)

    def chunk(j, carry):
        pltpu.async_copy(x0_h.at[ir_v.at[j]], buf.at[0], sem)
        pltpu.async_copy(x1_h.at[ir_v.at[j]], buf.at[1], sem)
        pltpu.async_copy(x2_h.at[ir_v.at[j]], buf.at[2], sem)
        pltpu.async_copy(x0_h.at[ic_v.at[j]], buf.at[3], sem)
        pltpu.async_copy(x1_h.at[ic_v.at[j]], buf.at[4], sem)
        cp = pltpu.async_copy(x2_h.at[ic_v.at[j]], buf.at[5], sem)
        for k in range(5):
            pltpu.make_async_copy(x0_h.at[ir_v.at[j]], buf.at[k], sem).wait()
        cp.wait()
        for g in range(BLK // 16):
            sl = pl.ds(g * 16, 16)
            d0 = buf[0, sl] - buf[3, sl]
            d1 = buf[1, sl] - buf[4, sl]
            d2 = buf[2, sl] - buf[5, sl]
            o_v[j, sl] = d0 * d0 + d1 * d1 + d2 * d2
        return carry

    lax.fori_loop(0, CW, chunk, 0)
    pltpu.sync_copy(o_v, out_h.at[wid])


GBLK = 64
GC = 79
GN = 4
TCW = GN * GC
TEW = TCW * GBLK
TROWS = 632


@functools.partial(
    pl.kernel,
    out_type=jax.ShapeDtypeStruct((NC, EPAD, D), jnp.float32),
    mesh=plsc.VectorSubcoreMesh(**_MESH),
    scratch_types=[
        pltpu.VMEM((2 * GC, GBLK), jnp.int32),
        pltpu.VMEM((3, GBLK, D), jnp.float32),
        pltpu.VMEM_SHARED((N, D), jnp.float32),
        pltpu.SemaphoreType.DMA,
        pltpu.SemaphoreType.DMA,
        pltpu.SemaphoreType.DMA,
    ],
)
def _gather_sc(hp_h, hq_h, idx_h, out_h, idx_v, buf, tab, sem_g, sem_o,
               sem_i):
    cid = lax.axis_index("c")
    sid = lax.axis_index("s")

    r0 = sid * TROWS
    pltpu.sync_copy(idx_h.at[cid, sid, 0], idx_v.at[pl.ds(0, GC)])

    @pl.when(jnp.logical_and(cid == 0, sid < NS - 1))
    def _stage_p():
        pltpu.sync_copy(hp_h.at[pl.ds(r0, TROWS)], tab.at[pl.ds(r0, TROWS)])

    @pl.when(jnp.logical_and(cid == 0, sid == NS - 1))
    def _stage_p_last():
        pltpu.sync_copy(hp_h.at[pl.ds((NS - 1) * TROWS, N - (NS - 1) * TROWS)],
                        tab.at[pl.ds((NS - 1) * TROWS, N - (NS - 1) * TROWS)])

    @pl.when(jnp.logical_and(cid == 1, sid < NS - 1))
    def _stage_q():
        pltpu.sync_copy(hq_h.at[pl.ds(r0, TROWS)], tab.at[pl.ds(r0, TROWS)])

    @pl.when(jnp.logical_and(cid == 1, sid == NS - 1))
    def _stage_q_last():
        pltpu.sync_copy(hq_h.at[pl.ds((NS - 1) * TROWS, N - (NS - 1) * TROWS)],
                        tab.at[pl.ds((NS - 1) * TROWS, N - (NS - 1) * TROWS)])

    plsc.subcore_barrier()

    base = sid * TEW
    pltpu.async_copy(idx_h.at[cid, sid, 1], idx_v.at[pl.ds(GC, GC)], sem_i)
    pltpu.async_copy(tab.at[idx_v.at[0]], buf.at[0], sem_g)
    pltpu.async_copy(tab.at[idx_v.at[1]], buf.at[1], sem_g)

    def _irow(j):
        g = lax.div(j, GC)
        return lax.rem(g, 2) * GC + lax.rem(j, GC)

    def chunk(j, carry):
        slot = lax.rem(j, 3)

        @pl.when(jnp.logical_and(lax.rem(j, GC) == GC - 3,
                                 lax.div(j, GC) < GN - 1))
        def _wait_idx():
            g = lax.div(j, GC) + 1
            pltpu.make_async_copy(
                idx_h.at[cid, sid, g],
                idx_v.at[pl.ds(lax.rem(g, 2) * GC, GC)], sem_i).wait()

        pltpu.make_async_copy(tab.at[idx_v.at[_irow(j)]], buf.at[slot],
                              sem_g).wait()

        @pl.when(jnp.logical_and(lax.rem(j, GC) == GC - 1,
                                 lax.div(j, GC) < GN - 2))
        def _load_idx():
            g = lax.div(j, GC) + 2
            pltpu.async_copy(idx_h.at[cid, sid, g],
                             idx_v.at[pl.ds(lax.rem(g, 2) * GC, GC)], sem_i)

        @pl.when(j >= 1)
        def _drain_prev_out():
            pltpu.make_async_copy(
                buf.at[lax.rem(j + 2, 3)],
                out_h.at[cid].at[pl.ds(base + (j - 1) * GBLK, GBLK)],
                sem_o).wait()

        pltpu.async_copy(buf.at[slot],
                         out_h.at[cid].at[pl.ds(base + j * GBLK, GBLK)],
                         sem_o)

        @pl.when(j + 2 < TCW)
        def _prefetch():
            pltpu.async_copy(tab.at[idx_v.at[_irow(j + 2)]],
                             buf.at[lax.rem(j + 2, 3)], sem_g)
        return carry

    lax.fori_loop(0, TCW, chunk, 0)
    pltpu.make_async_copy(
        buf.at[lax.rem(TCW - 1, 3)],
        out_h.at[cid].at[pl.ds(base + (TCW - 1) * GBLK, GBLK)],
        sem_o).wait()


@functools.partial(
    pl.kernel,
    out_type=jax.ShapeDtypeStruct((NC, NACC, D), jnp.float32),
    mesh=plsc.VectorSubcoreMesh(**_MESH),
    scratch_types=[
        pltpu.VMEM((SCW, SBLK), jnp.int32),
        pltpu.VMEM((2, SBLK, D), jnp.float32),
        pltpu.VMEM((ZR, D), jnp.float32),
        pltpu.VMEM_SHARED((NACC, D), jnp.float32),
        pltpu.SemaphoreType.DMA,
    ],
)
def _scatter_sc(ef2_h, idxs_h, out_h, idx_v, upd, zbuf, acc, sem):
    cid = lax.axis_index("c")
    sid = lax.axis_index("s")
    wid = sid * NC + cid

    def zrow(r, carry):
        for g in range(D // 16):
            zbuf[r, pl.ds(g * 16, 16)] = jnp.zeros((16,), jnp.float32)
        return carry

    lax.fori_loop(0, ZR, zrow, 0)

    base_r = sid * RPT
    for t in range(RPT // ZR):
        pltpu.sync_copy(zbuf, acc.at[pl.ds(base_r + t * ZR, ZR)])
    rem = RPT - (RPT // ZR) * ZR
    if rem:
        pltpu.sync_copy(zbuf.at[pl.ds(0, rem)],
                        acc.at[pl.ds(base_r + (RPT // ZR) * ZR, rem)])
    plsc.subcore_barrier()

    pltpu.sync_copy(idxs_h.at[wid], idx_v)
    base_e = wid * SEW
    pltpu.async_copy(ef2_h.at[pl.ds(base_e, SBLK)], upd.at[0], sem)

    def chunk(j, carry):
        slot = lax.rem(j, 2)
        nslot = lax.rem(j + 1, 2)

        @pl.when(j + 1 < SCW)
        def _prefetch():
            pltpu.async_copy(ef2_h.at[pl.ds(base_e + (j + 1) * SBLK, SBLK)],
                             upd.at[nslot], sem)

        pltpu.make_async_copy(ef2_h.at[pl.ds(base_e + j * SBLK, SBLK)],
                              upd.at[slot], sem).wait()
        pltpu.sync_copy(upd.at[slot], acc.at[idx_v.at[j]], add=True)
        return carry

    lax.fori_loop(0, SCW, chunk, 0)
    plsc.subcore_barrier()
    pltpu.sync_copy(acc.at[pl.ds(sid * RPT, RPT)],
                    out_h.at[cid].at[pl.ds(sid * RPT, RPT)])


def _silu(t):
    return t * jax.nn.sigmoid(t)


def _embed_body(h0_r, we_r, be_r, wa_r, wb_r, h_r, hp_r, hq_r):
    h = jnp.dot(h0_r[...], we_r[...],
                preferred_element_type=jnp.float32) + be_r[...]
    h_r[...] = h
    hp_r[...] = jnp.dot(h, wa_r[...], preferred_element_type=jnp.float32)
    hq_r[...] = jnp.dot(h, wb_r[...], preferred_element_type=jnp.float32)


def _edge_body(pre_r, ea8_r, weg_r, b1_r, w2_r, b2_r, out_r):
    f32 = jnp.float32
    t = pre_r[0] + pre_r[1] + jnp.dot(ea8_r[...], weg_r[...],
                                      preferred_element_type=f32) + b1_r[...]
    t = _silu(t)
    u = jnp.dot(t.astype(jnp.bfloat16), w2_r[...],
                preferred_element_type=f32) + b2_r[...]
    out_r[...] = _silu(u)


def _node_core(h, agg_r, h0_r, a_r, b_r, c_r, b1_r, w2_r, b2_r):
    f32 = jnp.float32
    agg = agg_r[0] + agg_r[1]
    t = (jnp.dot(h, a_r[...], preferred_element_type=f32)
         + jnp.dot(agg, b_r[...], preferred_element_type=f32)
         + jnp.dot(h0_r[...], c_r[...], preferred_element_type=f32)
         + b1_r[...])
    m = jnp.dot(_silu(t), w2_r[...], preferred_element_type=f32) + b2_r[...]
    return h + m


def _node_body(h_r, agg_r, h0_r, a_r, b_r, c_r, b1_r, w2_r, b2_r,
               wa_r, wb_r, hn_r, hp_r, hq_r):
    hn = _node_core(h_r[...], agg_r, h0_r, a_r, b_r, c_r, b1_r, w2_r, b2_r)
    hn_r[...] = hn
    hp_r[...] = jnp.dot(hn, wa_r[...], preferred_element_type=jnp.float32)
    hq_r[...] = jnp.dot(hn, wb_r[...], preferred_element_type=jnp.float32)


def _final_body(h_r, agg_r, h0_r, a_r, b_r, c_r, b1_r, w2_r, b2_r,
                ndw1_r, ndb1_r, ndw2_r, ndb2_r,
                gdw1_r, gdb1_r, gdw2_r, gdb2_r, out_r):
    f32 = jnp.float32
    hn = _node_core(h_r[...], agg_r, h0_r, a_r, b_r, c_r, b1_r, w2_r, b2_r)
    t = jnp.dot(_silu(jnp.dot(hn, ndw1_r[...], preferred_element_type=f32)
                      + ndb1_r[...]),
                ndw2_r[...], preferred_element_type=f32) + ndb2_r[...]
    u = _silu(jnp.dot(t, gdw1_r[...], preferred_element_type=f32)
              + gdb1_r[...])
    out_r[...] = jnp.dot(u, gdw2_r[...],
                         preferred_element_type=f32) + gdb2_r[...]


def _full(shape):
    return pl.BlockSpec(shape, lambda i: tuple(0 for _ in shape))


def _nblk():
    return pl.BlockSpec((BN, D), lambda i: (i, 0))


def _aggblk():
    return pl.BlockSpec((NC, BN, D), lambda i: (0, i, 0))


_W = _full((D, D))
_B = _full((1, D))


def _embed_call(h0, we, be, wa, wb):
    return pl.pallas_call(
        _embed_body,
        grid=(N // BN,),
        in_specs=[_nblk(), _W, _B, _W, _W],
        out_specs=[_nblk(), _nblk(), _nblk()],
        out_shape=[jax.ShapeDtypeStruct((N, D), jnp.float32)] * 3,
    )(h0, we, be, wa, wb)


def _edge_call(pre2, ea8, weg, b1, w2, b2):
    eblk = pl.BlockSpec((BE, D), lambda i: (i, 0))
    return pl.pallas_call(
        _edge_body,
        grid=(EPAD // BE,),
        in_specs=[pl.BlockSpec((NC, BE, D), lambda i: (0, i, 0)),
                  pl.BlockSpec((BE, 8), lambda i: (i, 0)),
                  _full((8, D)), _B, _W, _B],
        out_specs=eblk,
        out_shape=jax.ShapeDtypeStruct((EPAD, D), jnp.float32),
    )(pre2, ea8, weg, b1, w2.astype(jnp.bfloat16), b2)


def _node_call(h, aggp, h0, a, b, c, b1, w2, b2, wa, wb):
    return pl.pallas_call(
        _node_body,
        grid=(N // BN,),
        in_specs=[_nblk(), _aggblk(), _nblk(), _W, _W, _W, _B, _W, _B,
                  _W, _W],
        out_specs=[_nblk(), _nblk(), _nblk()],
        out_shape=[jax.ShapeDtypeStruct((N, D), jnp.float32)] * 3,
    )(h, aggp, h0, a, b, c, b1, w2, b2, wa, wb)


def _final_call(h, aggp, h0, a, b, c, b1, w2, b2, ndw1, ndb1, ndw2, ndb2,
                gdw1, gdb1, gdw2p, gdb2p):
    return pl.pallas_call(
        _final_body,
        grid=(N // BN,),
        in_specs=[_nblk(), _aggblk(), _nblk(), _W, _W, _W, _B, _W, _B,
                  _W, _B, _W, _B, _W, _B, _W, _B],
        out_specs=_nblk(),
        out_shape=jax.ShapeDtypeStruct((N, D), jnp.float32),
    )(h, aggp, h0, a, b, c, b1, w2, b2, ndw1, ndb1, ndw2, ndb2,
      gdw1, gdb1, gdw2p, gdb2p)


def kernel(h0, x, edges, edge_attr, emb_W, emb_b, e_W1, e_b1, e_W2, e_b2,
           n_W1, n_b1, n_W2, n_b2, nd_W1, nd_b1, nd_W2, nd_b2,
           gd_W1, gd_b1, gd_W2, gd_b2):
    f32 = jnp.float32
    row = edges[0]
    col = edges[1]
    pad = EPAD - E
    pidx = jnp.arange(pad, dtype=jnp.int32)
    row_gf = jnp.concatenate([row, pidx % N])
    col_gf = jnp.concatenate([col, (pidx * 7 + 3) % N])
    row_g = row_gf.reshape(NW, CW, BLK)
    col_g = col_gf.reshape(NW, CW, BLK)
    idx5 = jnp.stack([row_gf, col_gf]).reshape(2, NS, GN, GC, GBLK)
    row_s = jnp.concatenate([row, N + (pidx % SINK)]).reshape(NW, SCW, SBLK)

    ea_pad = jnp.pad(edge_attr, ((0, pad), (0, 0)))
    x0, x1, x2 = x[:, 0], x[:, 1], x[:, 2]

    wa = e_W1[:, :D, :]
    wb = e_W1[:, D:2 * D, :]
    wg = e_W1[:, 2 * D:, :]
    na = n_W1[:, :D, :]
    nb = n_W1[:, D:2 * D, :]
    nc_ = n_W1[:, 2 * D:, :]
    gdw2p = jnp.pad(gd_W2, ((0, 0), (0, D - OUT)))
    gdb2p = jnp.pad(gd_b2, (0, D - OUT))[None, :]

    h, hp, hq = _embed_call(h0, emb_W, emb_b[None, :], wa[0], wb[0])
    radial = _radial_sc(x0, x1, x2, row_g, col_g)
    ea8 = jnp.concatenate(
        [radial.reshape(EPAD, 1), ea_pad, jnp.zeros((EPAD, 3), f32)], axis=1)

    pred = None
    for i in range(L):
        weg = jnp.concatenate([wg[i], jnp.zeros((3, D), f32)], axis=0)
        pre2 = _gather_sc(hp, hq, idx5)
        ef2 = _edge_call(pre2, ea8, weg, e_b1[i][None, :], e_W2[i],
                         e_b2[i][None, :])
        aggp = _scatter_sc(ef2, row_s)
        if i < L - 1:
            h, hp, hq = _node_call(h, aggp, h0, na[i], nb[i], nc_[i],
                                   n_b1[i][None, :], n_W2[i],
                                   n_b2[i][None, :], wa[i + 1], wb[i + 1])
        else:
            pred = _final_call(h, aggp, h0, na[i], nb[i], nc_[i],
                               n_b1[i][None, :], n_W2[i], n_b2[i][None, :],
                               nd_W1, nd_b1[None, :], nd_W2, nd_b2[None, :],
                               gd_W1, gd_b1[None, :], gdw2p, gdb2p)
    return pred[:, :OUT]


# speedup vs baseline: 1.1002x; 1.0598x over previous
import functools

import jax
import jax.numpy as jnp
from jax import lax
from jax.experimental import pallas as pl
from jax.experimental.pallas import tpu as pltpu
from jax.experimental.pallas import tpu_sc as plsc

N = 10000
E = 320000
D = 128
DE = 4
L = 4
OUT = 21

NC = 2
NS = 16
NW = NC * NS

BLK = 128
CW = 79
EW = CW * BLK
EPAD = NW * EW

SINK = 112
NACC = N + SINK
RPT = NACC // NS
ZR = 16

SBLK = 128
SCW = 79
SEW = SCW * SBLK

BN = 2000
BE = 4096
TROWS = 632

_MESH = dict(core_axis_name="c", subcore_axis_name="s", num_cores=NC,
             num_subcores=NS)


def _wid():
    return lax.axis_index("s") * NC + lax.axis_index("c")


@functools.partial(
    pl.kernel,
    out_type=jax.ShapeDtypeStruct((NW, CW, BLK), jnp.float32),
    mesh=plsc.VectorSubcoreMesh(**_MESH),
    scratch_types=[
        pltpu.VMEM((CW, BLK), jnp.int32),
        pltpu.VMEM((CW, BLK), jnp.int32),
        pltpu.VMEM((6, BLK), jnp.float32),
        pltpu.VMEM((CW, BLK), jnp.float32),
        pltpu.VMEM_SHARED((N,), jnp.float32),
        pltpu.VMEM_SHARED((N,), jnp.float32),
        pltpu.VMEM_SHARED((N,), jnp.float32),
        pltpu.VMEM((TROWS,), jnp.float32),
        pltpu.SemaphoreType.DMA,
    ],
)
def _radial_sc(x0_h, x1_h, x2_h, idxr_h, idxc_h, out_h, ir_v, ic_v, buf,
               o_v, x0_s, x1_s, x2_s, stg, sem):
    wid = _wid()
    sid = lax.axis_index("s")
    pltpu.sync_copy(idxr_h.at[wid], ir_v)
    pltpu.sync_copy(idxc_h.at[wid], ic_v)

    r0 = sid * TROWS

    @pl.when(sid < NS - 1)
    def _stage():
        for xh, xs in ((x0_h, x0_s), (x1_h, x1_s), (x2_h, x2_s)):
            pltpu.sync_copy(xh.at[pl.ds(r0, TROWS)], stg)
            pltpu.sync_copy(stg, xs.at[pl.ds(r0, TROWS)])

    @pl.when(sid == NS - 1)
    def _stage_last():
        lo = (NS - 1) * TROWS
        for xh, xs in ((x0_h, x0_s), (x1_h, x1_s), (x2_h, x2_s)):
            pltpu.sync_copy(xh.at[pl.ds(lo, N - lo)], stg.at[pl.ds(0, N - lo)])
            pltpu.sync_copy(stg.at[pl.ds(0, N - lo)], xs.at[pl.ds(lo, N - lo)])

    plsc.subcore_barrier()

    def chunk(j, carry):
        pltpu.async_copy(x0_s.at[ir_v.at[j]], buf.at[0], sem)
        pltpu.async_copy(x1_s.at[ir_v.at[j]], buf.at[1], sem)
        pltpu.async_copy(x2_s.at[ir_v.at[j]], buf.at[2], sem)
        pltpu.async_copy(x0_s.at[ic_v.at[j]], buf.at[3], sem)
        pltpu.async_copy(x1_s.at[ic_v.at[j]], buf.at[4], sem)
        cp = pltpu.async_copy(x2_s.at[ic_v.at[j]], buf.at[5], sem)
        for k in range(5):
            pltpu.make_async_copy(x0_s.at[ir_v.at[j]], buf.at[k], sem).wait()
        cp.wait()
        for g in range(BLK // 16):
            sl = pl.ds(g * 16, 16)
            d0 = buf[0, sl] - buf[3, sl]
            d1 = buf[1, sl] - buf[4, sl]
            d2 = buf[2, sl] - buf[5, sl]
            o_v[j, sl] = d0 * d0 + d1 * d1 + d2 * d2
        return carry

    lax.fori_loop(0, CW, chunk, 0)
    pltpu.sync_copy(o_v, out_h.at[wid])


GBLK = 64
GC = 79
GN = 4
TCW = GN * GC
TEW = TCW * GBLK


@functools.partial(
    pl.kernel,
    out_type=jax.ShapeDtypeStruct((NC, EPAD, D), jnp.float32),
    mesh=plsc.VectorSubcoreMesh(**_MESH),
    scratch_types=[
        pltpu.VMEM((2 * GC, GBLK), jnp.int32),
        pltpu.VMEM((3, GBLK, D), jnp.float32),
        pltpu.VMEM_SHARED((N, D), jnp.float32),
        pltpu.SemaphoreType.DMA,
        pltpu.SemaphoreType.DMA,
        pltpu.SemaphoreType.DMA,
    ],
)
def _gather_sc(hp_h, hq_h, idx_h, out_h, idx_v, buf, tab, sem_g, sem_o,
               sem_i):
    cid = lax.axis_index("c")
    sid = lax.axis_index("s")

    r0 = sid * TROWS
    pltpu.sync_copy(idx_h.at[cid, sid, 0], idx_v.at[pl.ds(0, GC)])

    @pl.when(jnp.logical_and(cid == 0, sid < NS - 1))
    def _stage_p():
        pltpu.sync_copy(hp_h.at[pl.ds(r0, TROWS)], tab.at[pl.ds(r0, TROWS)])

    @pl.when(jnp.logical_and(cid == 0, sid == NS - 1))
    def _stage_p_last():
        pltpu.sync_copy(hp_h.at[pl.ds((NS - 1) * TROWS, N - (NS - 1) * TROWS)],
                        tab.at[pl.ds((NS - 1) * TROWS, N - (NS - 1) * TROWS)])

    @pl.when(jnp.logical_and(cid == 1, sid < NS - 1))
    def _stage_q():
        pltpu.sync_copy(hq_h.at[pl.ds(r0, TROWS)], tab.at[pl.ds(r0, TROWS)])

    @pl.when(jnp.logical_and(cid == 1, sid == NS - 1))
    def _stage_q_last():
        pltpu.sync_copy(hq_h.at[pl.ds((NS - 1) * TROWS, N - (NS - 1) * TROWS)],
                        tab.at[pl.ds((NS - 1) * TROWS, N - (NS - 1) * TROWS)])

    plsc.subcore_barrier()

    base = sid * TEW
    pltpu.async_copy(idx_h.at[cid, sid, 1], idx_v.at[pl.ds(GC, GC)], sem_i)
    pltpu.async_copy(tab.at[idx_v.at[0]], buf.at[0], sem_g)
    pltpu.async_copy(tab.at[idx_v.at[1]], buf.at[1], sem_g)

    def _irow(j):
        g = lax.div(j, GC)
        return lax.rem(g, 2) * GC + lax.rem(j, GC)

    def chunk(j, carry):
        slot = lax.rem(j, 3)

        @pl.when(jnp.logical_and(lax.rem(j, GC) == GC - 3,
                                 lax.div(j, GC) < GN - 1))
        def _wait_idx():
            g = lax.div(j, GC) + 1
            pltpu.make_async_copy(
                idx_h.at[cid, sid, g],
                idx_v.at[pl.ds(lax.rem(g, 2) * GC, GC)], sem_i).wait()

        pltpu.make_async_copy(tab.at[idx_v.at[_irow(j)]], buf.at[slot],
                              sem_g).wait()

        @pl.when(jnp.logical_and(lax.rem(j, GC) == GC - 1,
                                 lax.div(j, GC) < GN - 2))
        def _load_idx():
            g = lax.div(j, GC) + 2
            pltpu.async_copy(idx_h.at[cid, sid, g],
                             idx_v.at[pl.ds(lax.rem(g, 2) * GC, GC)], sem_i)

        @pl.when(j >= 1)
        def _drain_prev_out():
            pltpu.make_async_copy(
                buf.at[lax.rem(j + 2, 3)],
                out_h.at[cid].at[pl.ds(base + (j - 1) * GBLK, GBLK)],
                sem_o).wait()

        pltpu.async_copy(buf.at[slot],
                         out_h.at[cid].at[pl.ds(base + j * GBLK, GBLK)],
                         sem_o)

        @pl.when(j + 2 < TCW)
        def _prefetch():
            pltpu.async_copy(tab.at[idx_v.at[_irow(j + 2)]],
                             buf.at[lax.rem(j + 2, 3)], sem_g)
        return carry

    lax.fori_loop(0, TCW, chunk, 0)
    pltpu.make_async_copy(
        buf.at[lax.rem(TCW - 1, 3)],
        out_h.at[cid].at[pl.ds(base + (TCW - 1) * GBLK, GBLK)],
        sem_o).wait()


@functools.partial(
    pl.kernel,
    out_type=jax.ShapeDtypeStruct((NC, NACC, D), jnp.float32),
    mesh=plsc.VectorSubcoreMesh(**_MESH),
    scratch_types=[
        pltpu.VMEM((SCW, SBLK), jnp.int32),
        pltpu.VMEM((2, SBLK, D), jnp.float32),
        pltpu.VMEM((ZR, D), jnp.float32),
        pltpu.VMEM_SHARED((NACC, D), jnp.float32),
        pltpu.SemaphoreType.DMA,
    ],
)
def _scatter_sc(ef2_h, idxs_h, out_h, idx_v, upd, zbuf, acc, sem):
    cid = lax.axis_index("c")
    sid = lax.axis_index("s")
    wid = sid * NC + cid

    def zrow(r, carry):
        for g in range(D // 16):
            zbuf[r, pl.ds(g * 16, 16)] = jnp.zeros((16,), jnp.float32)
        return carry

    lax.fori_loop(0, ZR, zrow, 0)

    base_r = sid * RPT
    for t in range(RPT // ZR):
        pltpu.sync_copy(zbuf, acc.at[pl.ds(base_r + t * ZR, ZR)])
    rem = RPT - (RPT // ZR) * ZR
    if rem:
        pltpu.sync_copy(zbuf.at[pl.ds(0, rem)],
                        acc.at[pl.ds(base_r + (RPT // ZR) * ZR, rem)])
    plsc.subcore_barrier()

    pltpu.sync_copy(idxs_h.at[wid], idx_v)
    base_e = wid * SEW
    pltpu.async_copy(ef2_h.at[pl.ds(base_e, SBLK)], upd.at[0], sem)

    def chunk(j, carry):
        slot = lax.rem(j, 2)
        nslot = lax.rem(j + 1, 2)

        @pl.when(j + 1 < SCW)
        def _prefetch():
            pltpu.async_copy(ef2_h.at[pl.ds(base_e + (j + 1) * SBLK, SBLK)],
                             upd.at[nslot], sem)

        pltpu.make_async_copy(ef2_h.at[pl.ds(base_e + j * SBLK, SBLK)],
                              upd.at[slot], sem).wait()
        pltpu.sync_copy(upd.at[slot], acc.at[idx_v.at[j]], add=True)
        return carry

    lax.fori_loop(0, SCW, chunk, 0)
    plsc.subcore_barrier()
    pltpu.sync_copy(acc.at[pl.ds(sid * RPT, RPT)],
                    out_h.at[cid].at[pl.ds(sid * RPT, RPT)])


def _silu(t):
    return t * jax.nn.sigmoid(t)


def _embed_body(h0_r, we_r, be_r, wa_r, wb_r, h_r, hp_r, hq_r):
    h = jnp.dot(h0_r[...], we_r[...],
                preferred_element_type=jnp.float32) + be_r[...]
    h_r[...] = h
    hp_r[...] = jnp.dot(h, wa_r[...], preferred_element_type=jnp.float32)
    hq_r[...] = jnp.dot(h, wb_r[...], preferred_element_type=jnp.float32)


def _edge_body(pre_r, ea8_r, weg_r, b1_r, w2_r, b2_r, out_r):
    f32 = jnp.float32
    t = pre_r[0] + pre_r[1] + jnp.dot(ea8_r[...], weg_r[...],
                                      preferred_element_type=f32) + b1_r[...]
    t = _silu(t)
    u = jnp.dot(t.astype(jnp.bfloat16), w2_r[...],
                preferred_element_type=f32) + b2_r[...]
    out_r[...] = _silu(u)


def _node_core(h, agg_r, h0_r, a_r, b_r, c_r, b1_r, w2_r, b2_r):
    f32 = jnp.float32
    agg = agg_r[0] + agg_r[1]
    t = (jnp.dot(h, a_r[...], preferred_element_type=f32)
         + jnp.dot(agg, b_r[...], preferred_element_type=f32)
         + jnp.dot(h0_r[...], c_r[...], preferred_element_type=f32)
         + b1_r[...])
    m = jnp.dot(_silu(t), w2_r[...], preferred_element_type=f32) + b2_r[...]
    return h + m


def _node_body(h_r, agg_r, h0_r, a_r, b_r, c_r, b1_r, w2_r, b2_r,
               wa_r, wb_r, hn_r, hp_r, hq_r):
    hn = _node_core(h_r[...], agg_r, h0_r, a_r, b_r, c_r, b1_r, w2_r, b2_r)
    hn_r[...] = hn
    hp_r[...] = jnp.dot(hn, wa_r[...], preferred_element_type=jnp.float32)
    hq_r[...] = jnp.dot(hn, wb_r[...], preferred_element_type=jnp.float32)


def _final_body(h_r, agg_r, h0_r, a_r, b_r, c_r, b1_r, w2_r, b2_r,
                ndw1_r, ndb1_r, ndw2_r, ndb2_r,
                gdw1_r, gdb1_r, gdw2_r, gdb2_r, out_r):
    f32 = jnp.float32
    hn = _node_core(h_r[...], agg_r, h0_r, a_r, b_r, c_r, b1_r, w2_r, b2_r)
    t = jnp.dot(_silu(jnp.dot(hn, ndw1_r[...], preferred_element_type=f32)
                      + ndb1_r[...]),
                ndw2_r[...], preferred_element_type=f32) + ndb2_r[...]
    u = _silu(jnp.dot(t, gdw1_r[...], preferred_element_type=f32)
              + gdb1_r[...])
    out_r[...] = jnp.dot(u, gdw2_r[...],
                         preferred_element_type=f32) + gdb2_r[...]


def _full(shape):
    return pl.BlockSpec(shape, lambda i: tuple(0 for _ in shape))


def _nblk():
    return pl.BlockSpec((BN, D), lambda i: (i, 0))


def _aggblk():
    return pl.BlockSpec((NC, BN, D), lambda i: (0, i, 0))


_W = _full((D, D))
_B = _full((1, D))


def _embed_call(h0, we, be, wa, wb):
    return pl.pallas_call(
        _embed_body,
        grid=(N // BN,),
        in_specs=[_nblk(), _W, _B, _W, _W],
        out_specs=[_nblk(), _nblk(), _nblk()],
        out_shape=[jax.ShapeDtypeStruct((N, D), jnp.float32)] * 3,
    )(h0, we, be, wa, wb)


def _edge_call(pre2, ea8, weg, b1, w2, b2):
    eblk = pl.BlockSpec((BE, D), lambda i: (i, 0))
    return pl.pallas_call(
        _edge_body,
        grid=(EPAD // BE,),
        in_specs=[pl.BlockSpec((NC, BE, D), lambda i: (0, i, 0)),
                  pl.BlockSpec((BE, 8), lambda i: (i, 0)),
                  _full((8, D)), _B, _W, _B],
        out_specs=eblk,
        out_shape=jax.ShapeDtypeStruct((EPAD, D), jnp.float32),
    )(pre2, ea8, weg, b1, w2.astype(jnp.bfloat16), b2)


def _node_call(h, aggp, h0, a, b, c, b1, w2, b2, wa, wb):
    return pl.pallas_call(
        _node_body,
        grid=(N // BN,),
        in_specs=[_nblk(), _aggblk(), _nblk(), _W, _W, _W, _B, _W, _B,
                  _W, _W],
        out_specs=[_nblk(), _nblk(), _nblk()],
        out_shape=[jax.ShapeDtypeStruct((N, D), jnp.float32)] * 3,
    )(h, aggp, h0, a, b, c, b1, w2, b2, wa, wb)


def _final_call(h, aggp, h0, a, b, c, b1, w2, b2, ndw1, ndb1, ndw2, ndb2,
                gdw1, gdb1, gdw2p, gdb2p):
    return pl.pallas_call(
        _final_body,
        grid=(N // BN,),
        in_specs=[_nblk(), _aggblk(), _nblk(), _W, _W, _W, _B, _W, _B,
                  _W, _B, _W, _B, _W, _B, _W, _B],
        out_specs=_nblk(),
        out_shape=jax.ShapeDtypeStruct((N, D), jnp.float32),
    )(h, aggp, h0, a, b, c, b1, w2, b2, ndw1, ndb1, ndw2, ndb2,
      gdw1, gdb1, gdw2p, gdb2p)


def kernel(h0, x, edges, edge_attr, emb_W, emb_b, e_W1, e_b1, e_W2, e_b2,
           n_W1, n_b1, n_W2, n_b2, nd_W1, nd_b1, nd_W2, nd_b2,
           gd_W1, gd_b1, gd_W2, gd_b2):
    f32 = jnp.float32
    row = edges[0]
    col = edges[1]
    pad = EPAD - E
    pidx = jnp.arange(pad, dtype=jnp.int32)
    row_gf = jnp.concatenate([row, pidx % N])
    col_gf = jnp.concatenate([col, (pidx * 7 + 3) % N])
    row_g = row_gf.reshape(NW, CW, BLK)
    col_g = col_gf.reshape(NW, CW, BLK)
    idx5 = jnp.stack([row_gf, col_gf]).reshape(2, NS, GN, GC, GBLK)
    row_s = jnp.concatenate([row, N + (pidx % SINK)]).reshape(NW, SCW, SBLK)

    ea_pad = jnp.pad(edge_attr, ((0, pad), (0, 0)))
    x0, x1, x2 = x[:, 0], x[:, 1], x[:, 2]

    wa = e_W1[:, :D, :]
    wb = e_W1[:, D:2 * D, :]
    wg = e_W1[:, 2 * D:, :]
    na = n_W1[:, :D, :]
    nb = n_W1[:, D:2 * D, :]
    nc_ = n_W1[:, 2 * D:, :]
    gdw2p = jnp.pad(gd_W2, ((0, 0), (0, D - OUT)))
    gdb2p = jnp.pad(gd_b2, (0, D - OUT))[None, :]

    h, hp, hq = _embed_call(h0, emb_W, emb_b[None, :], wa[0], wb[0])
    radial = _radial_sc(x0, x1, x2, row_g, col_g)
    ea8 = jnp.concatenate(
        [radial.reshape(EPAD, 1), ea_pad, jnp.zeros((EPAD, 3), f32)], axis=1)

    pred = None
    for i in range(L):
        weg = jnp.concatenate([wg[i], jnp.zeros((3, D), f32)], axis=0)
        pre2 = _gather_sc(hp, hq, idx5)
        ef2 = _edge_call(pre2, ea8, weg, e_b1[i][None, :], e_W2[i],
                         e_b2[i][None, :])
        aggp = _scatter_sc(ef2, row_s)
        if i < L - 1:
            h, hp, hq = _node_call(h, aggp, h0, na[i], nb[i], nc_[i],
                                   n_b1[i][None, :], n_W2[i],
                                   n_b2[i][None, :], wa[i + 1], wb[i + 1])
        else:
            pred = _final_call(h, aggp, h0, na[i], nb[i], nc_[i],
                               n_b1[i][None, :], n_W2[i], n_b2[i][None, :],
                               nd_W1, nd_b1[None, :], nd_W2, nd_b2[None, :],
                               gd_W1, gd_b1[None, :], gdw2p, gdb2p)
    return pred[:, :OUT]


# speedup vs baseline: 1.1029x; 1.0025x over previous
import functools

import jax
import jax.numpy as jnp
from jax import lax
from jax.experimental import pallas as pl
from jax.experimental.pallas import tpu as pltpu
from jax.experimental.pallas import tpu_sc as plsc

N = 10000
E = 320000
D = 128
DE = 4
L = 4
OUT = 21

NC = 2
NS = 16
NW = NC * NS

BLK = 128
CW = 79
EW = CW * BLK
EPAD = NW * EW

SINK = 112
NACC = N + SINK
RPT = NACC // NS
ZR = 32

SBLK = 128
SCW = 79
SEW = SCW * SBLK

BN = 2000
BE = 4096
TROWS = 632

_MESH = dict(core_axis_name="c", subcore_axis_name="s", num_cores=NC,
             num_subcores=NS)


def _wid():
    return lax.axis_index("s") * NC + lax.axis_index("c")


@functools.partial(
    pl.kernel,
    out_type=jax.ShapeDtypeStruct((NW, CW, BLK), jnp.float32),
    mesh=plsc.VectorSubcoreMesh(**_MESH),
    scratch_types=[
        pltpu.VMEM((CW, BLK), jnp.int32),
        pltpu.VMEM((CW, BLK), jnp.int32),
        pltpu.VMEM((6, BLK), jnp.float32),
        pltpu.VMEM((CW, BLK), jnp.float32),
        pltpu.VMEM_SHARED((N,), jnp.float32),
        pltpu.VMEM_SHARED((N,), jnp.float32),
        pltpu.VMEM_SHARED((N,), jnp.float32),
        pltpu.VMEM((TROWS,), jnp.float32),
        pltpu.SemaphoreType.DMA,
    ],
)
def _radial_sc(x0_h, x1_h, x2_h, idxr_h, idxc_h, out_h, ir_v, ic_v, buf,
               o_v, x0_s, x1_s, x2_s, stg, sem):
    wid = _wid()
    sid = lax.axis_index("s")
    pltpu.sync_copy(idxr_h.at[wid], ir_v)
    pltpu.sync_copy(idxc_h.at[wid], ic_v)

    r0 = sid * TROWS

    @pl.when(sid < NS - 1)
    def _stage():
        for xh, xs in ((x0_h, x0_s), (x1_h, x1_s), (x2_h, x2_s)):
            pltpu.sync_copy(xh.at[pl.ds(r0, TROWS)], stg)
            pltpu.sync_copy(stg, xs.at[pl.ds(r0, TROWS)])

    @pl.when(sid == NS - 1)
    def _stage_last():
        lo = (NS - 1) * TROWS
        for xh, xs in ((x0_h, x0_s), (x1_h, x1_s), (x2_h, x2_s)):
            pltpu.sync_copy(xh.at[pl.ds(lo, N - lo)], stg.at[pl.ds(0, N - lo)])
            pltpu.sync_copy(stg.at[pl.ds(0, N - lo)], xs.at[pl.ds(lo, N - lo)])

    plsc.subcore_barrier()

    def chunk(j, carry):
        pltpu.async_copy(x0_s.at[ir_v.at[j]], buf.at[0], sem)
        pltpu.async_copy(x1_s.at[ir_v.at[j]], buf.at[1], sem)
        pltpu.async_copy(x2_s.at[ir_v.at[j]], buf.at[2], sem)
        pltpu.async_copy(x0_s.at[ic_v.at[j]], buf.at[3], sem)
        pltpu.async_copy(x1_s.at[ic_v.at[j]], buf.at[4], sem)
        cp = pltpu.async_copy(x2_s.at[ic_v.at[j]], buf.at[5], sem)
        for k in range(5):
            pltpu.make_async_copy(x0_s.at[ir_v.at[j]], buf.at[k], sem).wait()
        cp.wait()
        for g in range(BLK // 16):
            sl = pl.ds(g * 16, 16)
            d0 = buf[0, sl] - buf[3, sl]
            d1 = buf[1, sl] - buf[4, sl]
            d2 = buf[2, sl] - buf[5, sl]
            o_v[j, sl] = d0 * d0 + d1 * d1 + d2 * d2
        return carry

    lax.fori_loop(0, CW, chunk, 0)
    pltpu.sync_copy(o_v, out_h.at[wid])


GBLK = 64
GC = 79
GN = 4
TCW = GN * GC
TEW = TCW * GBLK


@functools.partial(
    pl.kernel,
    out_type=jax.ShapeDtypeStruct((NC, EPAD, D), jnp.float32),
    mesh=plsc.VectorSubcoreMesh(**_MESH),
    scratch_types=[
        pltpu.VMEM((2 * GC, GBLK), jnp.int32),
        pltpu.VMEM((3, GBLK, D), jnp.float32),
        pltpu.VMEM_SHARED((N, D), jnp.float32),
        pltpu.SemaphoreType.DMA,
        pltpu.SemaphoreType.DMA,
        pltpu.SemaphoreType.DMA,
    ],
)
def _gather_sc(hp_h, hq_h, idx_h, out_h, idx_v, buf, tab, sem_g, sem_o,
               sem_i):
    cid = lax.axis_index("c")
    sid = lax.axis_index("s")

    r0 = sid * TROWS
    pltpu.sync_copy(idx_h.at[cid, sid, 0], idx_v.at[pl.ds(0, GC)])

    @pl.when(jnp.logical_and(cid == 0, sid < NS - 1))
    def _stage_p():
        pltpu.sync_copy(hp_h.at[pl.ds(r0, TROWS)], tab.at[pl.ds(r0, TROWS)])

    @pl.when(jnp.logical_and(cid == 0, sid == NS - 1))
    def _stage_p_last():
        pltpu.sync_copy(hp_h.at[pl.ds((NS - 1) * TROWS, N - (NS - 1) * TROWS)],
                        tab.at[pl.ds((NS - 1) * TROWS, N - (NS - 1) * TROWS)])

    @pl.when(jnp.logical_and(cid == 1, sid < NS - 1))
    def _stage_q():
        pltpu.sync_copy(hq_h.at[pl.ds(r0, TROWS)], tab.at[pl.ds(r0, TROWS)])

    @pl.when(jnp.logical_and(cid == 1, sid == NS - 1))
    def _stage_q_last():
        pltpu.sync_copy(hq_h.at[pl.ds((NS - 1) * TROWS, N - (NS - 1) * TROWS)],
                        tab.at[pl.ds((NS - 1) * TROWS, N - (NS - 1) * TROWS)])

    plsc.subcore_barrier()

    base = sid * TEW
    pltpu.async_copy(idx_h.at[cid, sid, 1], idx_v.at[pl.ds(GC, GC)], sem_i)
    pltpu.async_copy(tab.at[idx_v.at[0]], buf.at[0], sem_g)
    pltpu.async_copy(tab.at[idx_v.at[1]], buf.at[1], sem_g)

    def _irow(j):
        g = lax.div(j, GC)
        return lax.rem(g, 2) * GC + lax.rem(j, GC)

    def chunk(j, carry):
        slot = lax.rem(j, 3)

        @pl.when(jnp.logical_and(lax.rem(j, GC) == GC - 3,
                                 lax.div(j, GC) < GN - 1))
        def _wait_idx():
            g = lax.div(j, GC) + 1
            pltpu.make_async_copy(
                idx_h.at[cid, sid, g],
                idx_v.at[pl.ds(lax.rem(g, 2) * GC, GC)], sem_i).wait()

        pltpu.make_async_copy(tab.at[idx_v.at[_irow(j)]], buf.at[slot],
                              sem_g).wait()

        @pl.when(jnp.logical_and(lax.rem(j, GC) == GC - 1,
                                 lax.div(j, GC) < GN - 2))
        def _load_idx():
            g = lax.div(j, GC) + 2
            pltpu.async_copy(idx_h.at[cid, sid, g],
                             idx_v.at[pl.ds(lax.rem(g, 2) * GC, GC)], sem_i)

        @pl.when(j >= 1)
        def _drain_prev_out():
            pltpu.make_async_copy(
                buf.at[lax.rem(j + 2, 3)],
                out_h.at[cid].at[pl.ds(base + (j - 1) * GBLK, GBLK)],
                sem_o).wait()

        pltpu.async_copy(buf.at[slot],
                         out_h.at[cid].at[pl.ds(base + j * GBLK, GBLK)],
                         sem_o)

        @pl.when(j + 2 < TCW)
        def _prefetch():
            pltpu.async_copy(tab.at[idx_v.at[_irow(j + 2)]],
                             buf.at[lax.rem(j + 2, 3)], sem_g)
        return carry

    lax.fori_loop(0, TCW, chunk, 0)
    pltpu.make_async_copy(
        buf.at[lax.rem(TCW - 1, 3)],
        out_h.at[cid].at[pl.ds(base + (TCW - 1) * GBLK, GBLK)],
        sem_o).wait()


@functools.partial(
    pl.kernel,
    out_type=jax.ShapeDtypeStruct((NC, NACC, D), jnp.float32),
    mesh=plsc.VectorSubcoreMesh(**_MESH),
    scratch_types=[
        pltpu.VMEM((SCW, SBLK), jnp.int32),
        pltpu.VMEM((2, SBLK, D), jnp.float32),
        pltpu.VMEM((ZR, D), jnp.float32),
        pltpu.VMEM_SHARED((NACC, D), jnp.float32),
        pltpu.SemaphoreType.DMA,
    ],
)
def _scatter_sc(ef2_h, idxs_h, out_h, idx_v, upd, zbuf, acc, sem):
    cid = lax.axis_index("c")
    sid = lax.axis_index("s")
    wid = sid * NC + cid

    def zrow(r, carry):
        for g in range(D // 16):
            zbuf[r, pl.ds(g * 16, 16)] = jnp.zeros((16,), jnp.float32)
        return carry

    lax.fori_loop(0, ZR, zrow, 0)

    base_r = sid * RPT
    for t in range(RPT // ZR):
        pltpu.sync_copy(zbuf, acc.at[pl.ds(base_r + t * ZR, ZR)])
    rem = RPT - (RPT // ZR) * ZR
    if rem:
        pltpu.sync_copy(zbuf.at[pl.ds(0, rem)],
                        acc.at[pl.ds(base_r + (RPT // ZR) * ZR, rem)])
    plsc.subcore_barrier()

    pltpu.sync_copy(idxs_h.at[wid], idx_v)
    base_e = wid * SEW
    pltpu.async_copy(ef2_h.at[pl.ds(base_e, SBLK)], upd.at[0], sem)

    def chunk(j, carry):
        slot = lax.rem(j, 2)
        nslot = lax.rem(j + 1, 2)

        @pl.when(j + 1 < SCW)
        def _prefetch():
            pltpu.async_copy(ef2_h.at[pl.ds(base_e + (j + 1) * SBLK, SBLK)],
                             upd.at[nslot], sem)

        pltpu.make_async_copy(ef2_h.at[pl.ds(base_e + j * SBLK, SBLK)],
                              upd.at[slot], sem).wait()
        pltpu.sync_copy(upd.at[slot], acc.at[idx_v.at[j]], add=True)
        return carry

    lax.fori_loop(0, SCW, chunk, 0)
    plsc.subcore_barrier()
    pltpu.sync_copy(acc.at[pl.ds(sid * RPT, RPT)],
                    out_h.at[cid].at[pl.ds(sid * RPT, RPT)])


def _silu(t):
    return t * jax.nn.sigmoid(t)


def _embed_body(h0_r, we_r, be_r, wa_r, wb_r, h_r, hp_r, hq_r):
    h = jnp.dot(h0_r[...], we_r[...],
                preferred_element_type=jnp.float32) + be_r[...]
    h_r[...] = h
    hp_r[...] = jnp.dot(h, wa_r[...], preferred_element_type=jnp.float32)
    hq_r[...] = jnp.dot(h, wb_r[...], preferred_element_type=jnp.float32)


def _edge_body(pre_r, ea8_r, weg_r, b1_r, w2_r, b2_r, out_r):
    f32 = jnp.float32
    t = pre_r[0] + pre_r[1] + jnp.dot(ea8_r[...], weg_r[...],
                                      preferred_element_type=f32) + b1_r[...]
    t = _silu(t)
    u = jnp.dot(t.astype(jnp.bfloat16), w2_r[...],
                preferred_element_type=f32) + b2_r[...]
    out_r[...] = _silu(u)


def _node_core(h, agg_r, h0_r, a_r, b_r, c_r, b1_r, w2_r, b2_r):
    f32 = jnp.float32
    agg = agg_r[0] + agg_r[1]
    t = (jnp.dot(h, a_r[...], preferred_element_type=f32)
         + jnp.dot(agg, b_r[...], preferred_element_type=f32)
         + jnp.dot(h0_r[...], c_r[...], preferred_element_type=f32)
         + b1_r[...])
    m = jnp.dot(_silu(t), w2_r[...], preferred_element_type=f32) + b2_r[...]
    return h + m


def _node_body(h_r, agg_r, h0_r, a_r, b_r, c_r, b1_r, w2_r, b2_r,
               wa_r, wb_r, hn_r, hp_r, hq_r):
    hn = _node_core(h_r[...], agg_r, h0_r, a_r, b_r, c_r, b1_r, w2_r, b2_r)
    hn_r[...] = hn
    hp_r[...] = jnp.dot(hn, wa_r[...], preferred_element_type=jnp.float32)
    hq_r[...] = jnp.dot(hn, wb_r[...], preferred_element_type=jnp.float32)


def _final_body(h_r, agg_r, h0_r, a_r, b_r, c_r, b1_r, w2_r, b2_r,
                ndw1_r, ndb1_r, ndw2_r, ndb2_r,
                gdw1_r, gdb1_r, gdw2_r, gdb2_r, out_r):
    f32 = jnp.float32
    hn = _node_core(h_r[...], agg_r, h0_r, a_r, b_r, c_r, b1_r, w2_r, b2_r)
    t = jnp.dot(_silu(jnp.dot(hn, ndw1_r[...], preferred_element_type=f32)
                      + ndb1_r[...]),
                ndw2_r[...], preferred_element_type=f32) + ndb2_r[...]
    u = _silu(jnp.dot(t, gdw1_r[...], preferred_element_type=f32)
              + gdb1_r[...])
    out_r[...] = jnp.dot(u, gdw2_r[...],
                         preferred_element_type=f32) + gdb2_r[...]


def _full(shape):
    return pl.BlockSpec(shape, lambda i: tuple(0 for _ in shape))


def _nblk():
    return pl.BlockSpec((BN, D), lambda i: (i, 0))


def _aggblk():
    return pl.BlockSpec((NC, BN, D), lambda i: (0, i, 0))


_W = _full((D, D))
_B = _full((1, D))


def _embed_call(h0, we, be, wa, wb):
    return pl.pallas_call(
        _embed_body,
        grid=(N // BN,),
        in_specs=[_nblk(), _W, _B, _W, _W],
        out_specs=[_nblk(), _nblk(), _nblk()],
        out_shape=[jax.ShapeDtypeStruct((N, D), jnp.float32)] * 3,
    )(h0, we, be, wa, wb)


def _edge_call(pre2, ea8, weg, b1, w2, b2):
    eblk = pl.BlockSpec((BE, D), lambda i: (i, 0))
    return pl.pallas_call(
        _edge_body,
        grid=(EPAD // BE,),
        in_specs=[pl.BlockSpec((NC, BE, D), lambda i: (0, i, 0)),
                  pl.BlockSpec((BE, 8), lambda i: (i, 0)),
                  _full((8, D)), _B, _W, _B],
        out_specs=eblk,
        out_shape=jax.ShapeDtypeStruct((EPAD, D), jnp.float32),
    )(pre2, ea8, weg, b1, w2.astype(jnp.bfloat16), b2)


def _node_call(h, aggp, h0, a, b, c, b1, w2, b2, wa, wb):
    return pl.pallas_call(
        _node_body,
        grid=(N // BN,),
        in_specs=[_nblk(), _aggblk(), _nblk(), _W, _W, _W, _B, _W, _B,
                  _W, _W],
        out_specs=[_nblk(), _nblk(), _nblk()],
        out_shape=[jax.ShapeDtypeStruct((N, D), jnp.float32)] * 3,
    )(h, aggp, h0, a, b, c, b1, w2, b2, wa, wb)


def _final_call(h, aggp, h0, a, b, c, b1, w2, b2, ndw1, ndb1, ndw2, ndb2,
                gdw1, gdb1, gdw2p, gdb2p):
    return pl.pallas_call(
        _final_body,
        grid=(N // BN,),
        in_specs=[_nblk(), _aggblk(), _nblk(), _W, _W, _W, _B, _W, _B,
                  _W, _B, _W, _B, _W, _B, _W, _B],
        out_specs=_nblk(),
        out_shape=jax.ShapeDtypeStruct((N, D), jnp.float32),
    )(h, aggp, h0, a, b, c, b1, w2, b2, ndw1, ndb1, ndw2, ndb2,
      gdw1, gdb1, gdw2p, gdb2p)


def kernel(h0, x, edges, edge_attr, emb_W, emb_b, e_W1, e_b1, e_W2, e_b2,
           n_W1, n_b1, n_W2, n_b2, nd_W1, nd_b1, nd_W2, nd_b2,
           gd_W1, gd_b1, gd_W2, gd_b2):
    f32 = jnp.float32
    row = edges[0]
    col = edges[1]
    pad = EPAD - E
    pidx = jnp.arange(pad, dtype=jnp.int32)
    row_gf = jnp.concatenate([row, pidx % N])
    col_gf = jnp.concatenate([col, (pidx * 7 + 3) % N])
    row_g = row_gf.reshape(NW, CW, BLK)
    col_g = col_gf.reshape(NW, CW, BLK)
    idx5 = jnp.stack([row_gf, col_gf]).reshape(2, NS, GN, GC, GBLK)
    row_s = jnp.concatenate([row, N + (pidx % SINK)]).reshape(NW, SCW, SBLK)

    ea_pad = jnp.pad(edge_attr, ((0, pad), (0, 0)))
    x0, x1, x2 = x[:, 0], x[:, 1], x[:, 2]

    wa = e_W1[:, :D, :]
    wb = e_W1[:, D:2 * D, :]
    wg = e_W1[:, 2 * D:, :]
    na = n_W1[:, :D, :]
    nb = n_W1[:, D:2 * D, :]
    nc_ = n_W1[:, 2 * D:, :]
    gdw2p = jnp.pad(gd_W2, ((0, 0), (0, D - OUT)))
    gdb2p = jnp.pad(gd_b2, (0, D - OUT))[None, :]

    h, hp, hq = _embed_call(h0, emb_W, emb_b[None, :], wa[0], wb[0])
    radial = _radial_sc(x0, x1, x2, row_g, col_g)
    ea8 = jnp.concatenate(
        [radial.reshape(EPAD, 1), ea_pad, jnp.zeros((EPAD, 3), f32)], axis=1)

    pred = None
    for i in range(L):
        weg = jnp.concatenate([wg[i], jnp.zeros((3, D), f32)], axis=0)
        pre2 = _gather_sc(hp, hq, idx5)
        ef2 = _edge_call(pre2, ea8, weg, e_b1[i][None, :], e_W2[i],
                         e_b2[i][None, :])
        aggp = _scatter_sc(ef2, row_s)
        if i < L - 1:
            h, hp, hq = _node_call(h, aggp, h0, na[i], nb[i], nc_[i],
                                   n_b1[i][None, :], n_W2[i],
                                   n_b2[i][None, :], wa[i + 1], wb[i + 1])
        else:
            pred = _final_call(h, aggp, h0, na[i], nb[i], nc_[i],
                               n_b1[i][None, :], n_W2[i], n_b2[i][None, :],
                               nd_W1, nd_b1[None, :], nd_W2, nd_b2[None, :],
                               gd_W1, gd_b1[None, :], gdw2p, gdb2p)
    return pred[:, :OUT]


# speedup vs baseline: 1.1133x; 1.0094x over previous
import functools

import jax
import jax.numpy as jnp
from jax import lax
from jax.experimental import pallas as pl
from jax.experimental.pallas import tpu as pltpu
from jax.experimental.pallas import tpu_sc as plsc

N = 10000
E = 320000
D = 128
DE = 4
L = 4
OUT = 21

NC = 2
NS = 16
NW = NC * NS

BLK = 128
CW = 79
EW = CW * BLK
EPAD = NW * EW

SINK = 112
NACC = N + SINK
RPT = NACC // NS
ZR = 32

SBLK = 128
SCW = 79
SEW = SCW * SBLK

BN = 2000
BE = 5056
TROWS = 632

_MESH = dict(core_axis_name="c", subcore_axis_name="s", num_cores=NC,
             num_subcores=NS)


def _wid():
    return lax.axis_index("s") * NC + lax.axis_index("c")


@functools.partial(
    pl.kernel,
    out_type=jax.ShapeDtypeStruct((NW, CW, BLK), jnp.float32),
    mesh=plsc.VectorSubcoreMesh(**_MESH),
    scratch_types=[
        pltpu.VMEM((CW, BLK), jnp.int32),
        pltpu.VMEM((CW, BLK), jnp.int32),
        pltpu.VMEM((6, BLK), jnp.float32),
        pltpu.VMEM((CW, BLK), jnp.float32),
        pltpu.VMEM_SHARED((N,), jnp.float32),
        pltpu.VMEM_SHARED((N,), jnp.float32),
        pltpu.VMEM_SHARED((N,), jnp.float32),
        pltpu.VMEM((TROWS,), jnp.float32),
        pltpu.SemaphoreType.DMA,
    ],
)
def _radial_sc(x0_h, x1_h, x2_h, idxr_h, idxc_h, out_h, ir_v, ic_v, buf,
               o_v, x0_s, x1_s, x2_s, stg, sem):
    wid = _wid()
    sid = lax.axis_index("s")
    pltpu.sync_copy(idxr_h.at[wid], ir_v)
    pltpu.sync_copy(idxc_h.at[wid], ic_v)

    r0 = sid * TROWS

    @pl.when(sid < NS - 1)
    def _stage():
        for xh, xs in ((x0_h, x0_s), (x1_h, x1_s), (x2_h, x2_s)):
            pltpu.sync_copy(xh.at[pl.ds(r0, TROWS)], stg)
            pltpu.sync_copy(stg, xs.at[pl.ds(r0, TROWS)])

    @pl.when(sid == NS - 1)
    def _stage_last():
        lo = (NS - 1) * TROWS
        for xh, xs in ((x0_h, x0_s), (x1_h, x1_s), (x2_h, x2_s)):
            pltpu.sync_copy(xh.at[pl.ds(lo, N - lo)], stg.at[pl.ds(0, N - lo)])
            pltpu.sync_copy(stg.at[pl.ds(0, N - lo)], xs.at[pl.ds(lo, N - lo)])

    plsc.subcore_barrier()

    def chunk(j, carry):
        pltpu.async_copy(x0_s.at[ir_v.at[j]], buf.at[0], sem)
        pltpu.async_copy(x1_s.at[ir_v.at[j]], buf.at[1], sem)
        pltpu.async_copy(x2_s.at[ir_v.at[j]], buf.at[2], sem)
        pltpu.async_copy(x0_s.at[ic_v.at[j]], buf.at[3], sem)
        pltpu.async_copy(x1_s.at[ic_v.at[j]], buf.at[4], sem)
        cp = pltpu.async_copy(x2_s.at[ic_v.at[j]], buf.at[5], sem)
        for k in range(5):
            pltpu.make_async_copy(x0_s.at[ir_v.at[j]], buf.at[k], sem).wait()
        cp.wait()
        for g in range(BLK // 16):
            sl = pl.ds(g * 16, 16)
            d0 = buf[0, sl] - buf[3, sl]
            d1 = buf[1, sl] - buf[4, sl]
            d2 = buf[2, sl] - buf[5, sl]
            o_v[j, sl] = d0 * d0 + d1 * d1 + d2 * d2
        return carry

    lax.fori_loop(0, CW, chunk, 0)
    pltpu.sync_copy(o_v, out_h.at[wid])


GBLK = 64
GC = 79
GN = 4
TCW = GN * GC
TEW = TCW * GBLK


@functools.partial(
    pl.kernel,
    out_type=jax.ShapeDtypeStruct((NC, EPAD, D), jnp.float32),
    mesh=plsc.VectorSubcoreMesh(**_MESH),
    scratch_types=[
        pltpu.VMEM((2 * GC, GBLK), jnp.int32),
        pltpu.VMEM((3, GBLK, D), jnp.float32),
        pltpu.VMEM_SHARED((N, D), jnp.float32),
        pltpu.SemaphoreType.DMA,
        pltpu.SemaphoreType.DMA,
        pltpu.SemaphoreType.DMA,
    ],
)
def _gather_sc(hp_h, hq_h, idx_h, out_h, idx_v, buf, tab, sem_g, sem_o,
               sem_i):
    cid = lax.axis_index("c")
    sid = lax.axis_index("s")

    r0 = sid * TROWS
    pltpu.sync_copy(idx_h.at[cid, sid, 0], idx_v.at[pl.ds(0, GC)])

    @pl.when(jnp.logical_and(cid == 0, sid < NS - 1))
    def _stage_p():
        pltpu.sync_copy(hp_h.at[pl.ds(r0, TROWS)], tab.at[pl.ds(r0, TROWS)])

    @pl.when(jnp.logical_and(cid == 0, sid == NS - 1))
    def _stage_p_last():
        pltpu.sync_copy(hp_h.at[pl.ds((NS - 1) * TROWS, N - (NS - 1) * TROWS)],
                        tab.at[pl.ds((NS - 1) * TROWS, N - (NS - 1) * TROWS)])

    @pl.when(jnp.logical_and(cid == 1, sid < NS - 1))
    def _stage_q():
        pltpu.sync_copy(hq_h.at[pl.ds(r0, TROWS)], tab.at[pl.ds(r0, TROWS)])

    @pl.when(jnp.logical_and(cid == 1, sid == NS - 1))
    def _stage_q_last():
        pltpu.sync_copy(hq_h.at[pl.ds((NS - 1) * TROWS, N - (NS - 1) * TROWS)],
                        tab.at[pl.ds((NS - 1) * TROWS, N - (NS - 1) * TROWS)])

    plsc.subcore_barrier()

    base = sid * TEW
    pltpu.async_copy(idx_h.at[cid, sid, 1], idx_v.at[pl.ds(GC, GC)], sem_i)
    pltpu.async_copy(tab.at[idx_v.at[0]], buf.at[0], sem_g)
    pltpu.async_copy(tab.at[idx_v.at[1]], buf.at[1], sem_g)

    def _irow(j):
        g = lax.div(j, GC)
        return lax.rem(g, 2) * GC + lax.rem(j, GC)

    def chunk(j, carry):
        slot = lax.rem(j, 3)

        @pl.when(jnp.logical_and(lax.rem(j, GC) == GC - 3,
                                 lax.div(j, GC) < GN - 1))
        def _wait_idx():
            g = lax.div(j, GC) + 1
            pltpu.make_async_copy(
                idx_h.at[cid, sid, g],
                idx_v.at[pl.ds(lax.rem(g, 2) * GC, GC)], sem_i).wait()

        pltpu.make_async_copy(tab.at[idx_v.at[_irow(j)]], buf.at[slot],
                              sem_g).wait()

        @pl.when(jnp.logical_and(lax.rem(j, GC) == GC - 1,
                                 lax.div(j, GC) < GN - 2))
        def _load_idx():
            g = lax.div(j, GC) + 2
            pltpu.async_copy(idx_h.at[cid, sid, g],
                             idx_v.at[pl.ds(lax.rem(g, 2) * GC, GC)], sem_i)

        @pl.when(j >= 1)
        def _drain_prev_out():
            pltpu.make_async_copy(
                buf.at[lax.rem(j + 2, 3)],
                out_h.at[cid].at[pl.ds(base + (j - 1) * GBLK, GBLK)],
                sem_o).wait()

        pltpu.async_copy(buf.at[slot],
                         out_h.at[cid].at[pl.ds(base + j * GBLK, GBLK)],
                         sem_o)

        @pl.when(j + 2 < TCW)
        def _prefetch():
            pltpu.async_copy(tab.at[idx_v.at[_irow(j + 2)]],
                             buf.at[lax.rem(j + 2, 3)], sem_g)
        return carry

    lax.fori_loop(0, TCW, chunk, 0)
    pltpu.make_async_copy(
        buf.at[lax.rem(TCW - 1, 3)],
        out_h.at[cid].at[pl.ds(base + (TCW - 1) * GBLK, GBLK)],
        sem_o).wait()


@functools.partial(
    pl.kernel,
    out_type=jax.ShapeDtypeStruct((NC, NACC, D), jnp.float32),
    mesh=plsc.VectorSubcoreMesh(**_MESH),
    scratch_types=[
        pltpu.VMEM((SCW, SBLK), jnp.int32),
        pltpu.VMEM((2, SBLK, D), jnp.float32),
        pltpu.VMEM((ZR, D), jnp.float32),
        pltpu.VMEM_SHARED((NACC, D), jnp.float32),
        pltpu.SemaphoreType.DMA,
    ],
)
def _scatter_sc(ef2_h, idxs_h, out_h, idx_v, upd, zbuf, acc, sem):
    cid = lax.axis_index("c")
    sid = lax.axis_index("s")
    wid = sid * NC + cid

    def zrow(r, carry):
        for g in range(D // 16):
            zbuf[r, pl.ds(g * 16, 16)] = jnp.zeros((16,), jnp.float32)
        return carry

    lax.fori_loop(0, ZR, zrow, 0)

    base_r = sid * RPT
    for t in range(RPT // ZR):
        pltpu.sync_copy(zbuf, acc.at[pl.ds(base_r + t * ZR, ZR)])
    rem = RPT - (RPT // ZR) * ZR
    if rem:
        pltpu.sync_copy(zbuf.at[pl.ds(0, rem)],
                        acc.at[pl.ds(base_r + (RPT // ZR) * ZR, rem)])
    plsc.subcore_barrier()

    pltpu.sync_copy(idxs_h.at[wid], idx_v)
    base_e = wid * SEW
    pltpu.async_copy(ef2_h.at[pl.ds(base_e, SBLK)], upd.at[0], sem)

    def chunk(j, carry):
        slot = lax.rem(j, 2)
        nslot = lax.rem(j + 1, 2)

        @pl.when(j + 1 < SCW)
        def _prefetch():
            pltpu.async_copy(ef2_h.at[pl.ds(base_e + (j + 1) * SBLK, SBLK)],
                             upd.at[nslot], sem)

        pltpu.make_async_copy(ef2_h.at[pl.ds(base_e + j * SBLK, SBLK)],
                              upd.at[slot], sem).wait()
        pltpu.sync_copy(upd.at[slot], acc.at[idx_v.at[j]], add=True)
        return carry

    lax.fori_loop(0, SCW, chunk, 0)
    plsc.subcore_barrier()
    pltpu.sync_copy(acc.at[pl.ds(sid * RPT, RPT)],
                    out_h.at[cid].at[pl.ds(sid * RPT, RPT)])


def _silu(t):
    return t * jax.nn.sigmoid(t)


def _embed_body(h0_r, we_r, be_r, wa_r, wb_r, h_r, hp_r, hq_r):
    h = jnp.dot(h0_r[...], we_r[...],
                preferred_element_type=jnp.float32) + be_r[...]
    h_r[...] = h
    hp_r[...] = jnp.dot(h, wa_r[...], preferred_element_type=jnp.float32)
    hq_r[...] = jnp.dot(h, wb_r[...], preferred_element_type=jnp.float32)


def _edge_body(pre_r, ea8_r, weg_r, b1_r, w2_r, b2_r, out_r):
    f32 = jnp.float32
    t = pre_r[0] + pre_r[1] + jnp.dot(ea8_r[...], weg_r[...],
                                      preferred_element_type=f32) + b1_r[...]
    t = _silu(t)
    u = jnp.dot(t.astype(jnp.bfloat16), w2_r[...],
                preferred_element_type=f32) + b2_r[...]
    out_r[...] = _silu(u)


def _node_core(h, agg_r, h0_r, a_r, b_r, c_r, b1_r, w2_r, b2_r):
    f32 = jnp.float32
    agg = agg_r[0] + agg_r[1]
    t = (jnp.dot(h, a_r[...], preferred_element_type=f32)
         + jnp.dot(agg, b_r[...], preferred_element_type=f32)
         + jnp.dot(h0_r[...], c_r[...], preferred_element_type=f32)
         + b1_r[...])
    m = jnp.dot(_silu(t), w2_r[...], preferred_element_type=f32) + b2_r[...]
    return h + m


def _node_body(h_r, agg_r, h0_r, a_r, b_r, c_r, b1_r, w2_r, b2_r,
               wa_r, wb_r, hn_r, hp_r, hq_r):
    hn = _node_core(h_r[...], agg_r, h0_r, a_r, b_r, c_r, b1_r, w2_r, b2_r)
    hn_r[...] = hn
    hp_r[...] = jnp.dot(hn, wa_r[...], preferred_element_type=jnp.float32)
    hq_r[...] = jnp.dot(hn, wb_r[...], preferred_element_type=jnp.float32)


def _final_body(h_r, agg_r, h0_r, a_r, b_r, c_r, b1_r, w2_r, b2_r,
                ndw1_r, ndb1_r, ndw2_r, ndb2_r,
                gdw1_r, gdb1_r, gdw2_r, gdb2_r, out_r):
    f32 = jnp.float32
    hn = _node_core(h_r[...], agg_r, h0_r, a_r, b_r, c_r, b1_r, w2_r, b2_r)
    t = jnp.dot(_silu(jnp.dot(hn, ndw1_r[...], preferred_element_type=f32)
                      + ndb1_r[...]),
                ndw2_r[...], preferred_element_type=f32) + ndb2_r[...]
    u = _silu(jnp.dot(t, gdw1_r[...], preferred_element_type=f32)
              + gdb1_r[...])
    out_r[...] = jnp.dot(u, gdw2_r[...],
                         preferred_element_type=f32) + gdb2_r[...]


def _full(shape):
    return pl.BlockSpec(shape, lambda i: tuple(0 for _ in shape))


def _nblk():
    return pl.BlockSpec((BN, D), lambda i: (i, 0))


def _aggblk():
    return pl.BlockSpec((NC, BN, D), lambda i: (0, i, 0))


_W = _full((D, D))
_B = _full((1, D))


def _embed_call(h0, we, be, wa, wb):
    return pl.pallas_call(
        _embed_body,
        grid=(N // BN,),
        in_specs=[_nblk(), _W, _B, _W, _W],
        out_specs=[_nblk(), _nblk(), _nblk()],
        out_shape=[jax.ShapeDtypeStruct((N, D), jnp.float32)] * 3,
    )(h0, we, be, wa, wb)


def _edge_call(pre2, ea8, weg, b1, w2, b2):
    eblk = pl.BlockSpec((BE, D), lambda i: (i, 0))
    return pl.pallas_call(
        _edge_body,
        grid=(EPAD // BE,),
        in_specs=[pl.BlockSpec((NC, BE, D), lambda i: (0, i, 0)),
                  pl.BlockSpec((BE, 8), lambda i: (i, 0)),
                  _full((8, D)), _B, _W, _B],
        out_specs=eblk,
        out_shape=jax.ShapeDtypeStruct((EPAD, D), jnp.float32),
    )(pre2, ea8, weg, b1, w2.astype(jnp.bfloat16), b2)


def _node_call(h, aggp, h0, a, b, c, b1, w2, b2, wa, wb):
    return pl.pallas_call(
        _node_body,
        grid=(N // BN,),
        in_specs=[_nblk(), _aggblk(), _nblk(), _W, _W, _W, _B, _W, _B,
                  _W, _W],
        out_specs=[_nblk(), _nblk(), _nblk()],
        out_shape=[jax.ShapeDtypeStruct((N, D), jnp.float32)] * 3,
    )(h, aggp, h0, a, b, c, b1, w2, b2, wa, wb)


def _final_call(h, aggp, h0, a, b, c, b1, w2, b2, ndw1, ndb1, ndw2, ndb2,
                gdw1, gdb1, gdw2p, gdb2p):
    return pl.pallas_call(
        _final_body,
        grid=(N // BN,),
        in_specs=[_nblk(), _aggblk(), _nblk(), _W, _W, _W, _B, _W, _B,
                  _W, _B, _W, _B, _W, _B, _W, _B],
        out_specs=_nblk(),
        out_shape=jax.ShapeDtypeStruct((N, D), jnp.float32),
    )(h, aggp, h0, a, b, c, b1, w2, b2, ndw1, ndb1, ndw2, ndb2,
      gdw1, gdb1, gdw2p, gdb2p)


def kernel(h0, x, edges, edge_attr, emb_W, emb_b, e_W1, e_b1, e_W2, e_b2,
           n_W1, n_b1, n_W2, n_b2, nd_W1, nd_b1, nd_W2, nd_b2,
           gd_W1, gd_b1, gd_W2, gd_b2):
    f32 = jnp.float32
    row = edges[0]
    col = edges[1]
    pad = EPAD - E
    pidx = jnp.arange(pad, dtype=jnp.int32)
    row_gf = jnp.concatenate([row, pidx % N])
    col_gf = jnp.concatenate([col, (pidx * 7 + 3) % N])
    row_g = row_gf.reshape(NW, CW, BLK)
    col_g = col_gf.reshape(NW, CW, BLK)
    idx5 = jnp.stack([row_gf, col_gf]).reshape(2, NS, GN, GC, GBLK)
    row_s = jnp.concatenate([row, N + (pidx % SINK)]).reshape(NW, SCW, SBLK)

    ea_pad = jnp.pad(edge_attr, ((0, pad), (0, 0)))
    x0, x1, x2 = x[:, 0], x[:, 1], x[:, 2]

    wa = e_W1[:, :D, :]
    wb = e_W1[:, D:2 * D, :]
    wg = e_W1[:, 2 * D:, :]
    na = n_W1[:, :D, :]
    nb = n_W1[:, D:2 * D, :]
    nc_ = n_W1[:, 2 * D:, :]
    gdw2p = jnp.pad(gd_W2, ((0, 0), (0, D - OUT)))
    gdb2p = jnp.pad(gd_b2, (0, D - OUT))[None, :]

    h, hp, hq = _embed_call(h0, emb_W, emb_b[None, :], wa[0], wb[0])
    radial = _radial_sc(x0, x1, x2, row_g, col_g)
    ea8 = jnp.concatenate(
        [radial.reshape(EPAD, 1), ea_pad, jnp.zeros((EPAD, 3), f32)], axis=1)

    pred = None
    for i in range(L):
        weg = jnp.concatenate([wg[i], jnp.zeros((3, D), f32)], axis=0)
        pre2 = _gather_sc(hp, hq, idx5)
        ef2 = _edge_call(pre2, ea8, weg, e_b1[i][None, :], e_W2[i],
                         e_b2[i][None, :])
        aggp = _scatter_sc(ef2, row_s)
        if i < L - 1:
            h, hp, hq = _node_call(h, aggp, h0, na[i], nb[i], nc_[i],
                                   n_b1[i][None, :], n_W2[i],
                                   n_b2[i][None, :], wa[i + 1], wb[i + 1])
        else:
            pred = _final_call(h, aggp, h0, na[i], nb[i], nc_[i],
                               n_b1[i][None, :], n_W2[i], n_b2[i][None, :],
                               nd_W1, nd_b1[None, :], nd_W2, nd_b2[None, :],
                               gd_W1, gd_b1[None, :], gdw2p, gdb2p)
    return pred[:, :OUT]


# speedup vs baseline: 1.1176x; 1.0039x over previous
import functools

import jax
import jax.numpy as jnp
from jax import lax
from jax.experimental import pallas as pl
from jax.experimental.pallas import tpu as pltpu
from jax.experimental.pallas import tpu_sc as plsc

N = 10000
E = 320000
D = 128
DE = 4
L = 4
OUT = 21

NC = 2
NS = 16
NW = NC * NS

BLK = 128
CW = 79
EW = CW * BLK
EPAD = NW * EW

SINK = 112
NACC = N + SINK
RPT = NACC // NS
ZR = 32

SBLK = 128
SCW = 79
SEW = SCW * SBLK

BN = 2000
BE = 5056
TROWS = 632

_MESH = dict(core_axis_name="c", subcore_axis_name="s", num_cores=NC,
             num_subcores=NS)


def _wid():
    return lax.axis_index("s") * NC + lax.axis_index("c")


@functools.partial(
    pl.kernel,
    out_type=jax.ShapeDtypeStruct((NW, CW, BLK), jnp.float32),
    mesh=plsc.VectorSubcoreMesh(**_MESH),
    scratch_types=[
        pltpu.VMEM((CW, BLK), jnp.int32),
        pltpu.VMEM((CW, BLK), jnp.int32),
        pltpu.VMEM((6, BLK), jnp.float32),
        pltpu.VMEM((CW, BLK), jnp.float32),
        pltpu.VMEM_SHARED((N,), jnp.float32),
        pltpu.VMEM_SHARED((N,), jnp.float32),
        pltpu.VMEM_SHARED((N,), jnp.float32),
        pltpu.VMEM((TROWS,), jnp.float32),
        pltpu.SemaphoreType.DMA,
    ],
)
def _radial_sc(x0_h, x1_h, x2_h, idxr_h, idxc_h, out_h, ir_v, ic_v, buf,
               o_v, x0_s, x1_s, x2_s, stg, sem):
    wid = _wid()
    sid = lax.axis_index("s")
    pltpu.sync_copy(idxr_h.at[wid], ir_v)
    pltpu.sync_copy(idxc_h.at[wid], ic_v)

    r0 = sid * TROWS

    @pl.when(sid < NS - 1)
    def _stage():
        for xh, xs in ((x0_h, x0_s), (x1_h, x1_s), (x2_h, x2_s)):
            pltpu.sync_copy(xh.at[pl.ds(r0, TROWS)], stg)
            pltpu.sync_copy(stg, xs.at[pl.ds(r0, TROWS)])

    @pl.when(sid == NS - 1)
    def _stage_last():
        lo = (NS - 1) * TROWS
        for xh, xs in ((x0_h, x0_s), (x1_h, x1_s), (x2_h, x2_s)):
            pltpu.sync_copy(xh.at[pl.ds(lo, N - lo)], stg.at[pl.ds(0, N - lo)])
            pltpu.sync_copy(stg.at[pl.ds(0, N - lo)], xs.at[pl.ds(lo, N - lo)])

    plsc.subcore_barrier()

    def chunk(j, carry):
        pltpu.async_copy(x0_s.at[ir_v.at[j]], buf.at[0], sem)
        pltpu.async_copy(x1_s.at[ir_v.at[j]], buf.at[1], sem)
        pltpu.async_copy(x2_s.at[ir_v.at[j]], buf.at[2], sem)
        pltpu.async_copy(x0_s.at[ic_v.at[j]], buf.at[3], sem)
        pltpu.async_copy(x1_s.at[ic_v.at[j]], buf.at[4], sem)
        cp = pltpu.async_copy(x2_s.at[ic_v.at[j]], buf.at[5], sem)
        for k in range(5):
            pltpu.make_async_copy(x0_s.at[ir_v.at[j]], buf.at[k], sem).wait()
        cp.wait()
        for g in range(BLK // 16):
            sl = pl.ds(g * 16, 16)
            d0 = buf[0, sl] - buf[3, sl]
            d1 = buf[1, sl] - buf[4, sl]
            d2 = buf[2, sl] - buf[5, sl]
            o_v[j, sl] = d0 * d0 + d1 * d1 + d2 * d2
        return carry

    lax.fori_loop(0, CW, chunk, 0)
    pltpu.sync_copy(o_v, out_h.at[wid])


GBLK = 64
GC = 79
GN = 4
TCW = GN * GC
TEW = TCW * GBLK


@functools.partial(
    pl.kernel,
    out_type=jax.ShapeDtypeStruct((NC, EPAD, D), jnp.float32),
    mesh=plsc.VectorSubcoreMesh(**_MESH),
    scratch_types=[
        pltpu.VMEM((2 * GC, GBLK), jnp.int32),
        pltpu.VMEM((3, GBLK, D), jnp.float32),
        pltpu.VMEM_SHARED((N, D), jnp.float32),
        pltpu.SemaphoreType.DMA,
        pltpu.SemaphoreType.DMA,
        pltpu.SemaphoreType.DMA,
    ],
)
def _gather_sc(hp_h, hq_h, idx_h, out_h, idx_v, buf, tab, sem_g, sem_o,
               sem_i):
    cid = lax.axis_index("c")
    sid = lax.axis_index("s")

    r0 = sid * TROWS
    pltpu.sync_copy(idx_h.at[cid, sid, 0], idx_v.at[pl.ds(0, GC)])

    @pl.when(jnp.logical_and(cid == 0, sid < NS - 1))
    def _stage_p():
        pltpu.sync_copy(hp_h.at[pl.ds(r0, TROWS)], tab.at[pl.ds(r0, TROWS)])

    @pl.when(jnp.logical_and(cid == 0, sid == NS - 1))
    def _stage_p_last():
        pltpu.sync_copy(hp_h.at[pl.ds((NS - 1) * TROWS, N - (NS - 1) * TROWS)],
                        tab.at[pl.ds((NS - 1) * TROWS, N - (NS - 1) * TROWS)])

    @pl.when(jnp.logical_and(cid == 1, sid < NS - 1))
    def _stage_q():
        pltpu.sync_copy(hq_h.at[pl.ds(r0, TROWS)], tab.at[pl.ds(r0, TROWS)])

    @pl.when(jnp.logical_and(cid == 1, sid == NS - 1))
    def _stage_q_last():
        pltpu.sync_copy(hq_h.at[pl.ds((NS - 1) * TROWS, N - (NS - 1) * TROWS)],
                        tab.at[pl.ds((NS - 1) * TROWS, N - (NS - 1) * TROWS)])

    plsc.subcore_barrier()

    base = sid * TEW
    pltpu.async_copy(idx_h.at[cid, sid, 1], idx_v.at[pl.ds(GC, GC)], sem_i)
    pltpu.async_copy(tab.at[idx_v.at[0]], buf.at[0], sem_g)
    pltpu.async_copy(tab.at[idx_v.at[1]], buf.at[1], sem_g)

    def _irow(j):
        g = lax.div(j, GC)
        return lax.rem(g, 2) * GC + lax.rem(j, GC)

    def chunk(j, carry):
        slot = lax.rem(j, 3)

        @pl.when(jnp.logical_and(lax.rem(j, GC) == GC - 3,
                                 lax.div(j, GC) < GN - 1))
        def _wait_idx():
            g = lax.div(j, GC) + 1
            pltpu.make_async_copy(
                idx_h.at[cid, sid, g],
                idx_v.at[pl.ds(lax.rem(g, 2) * GC, GC)], sem_i).wait()

        pltpu.make_async_copy(tab.at[idx_v.at[_irow(j)]], buf.at[slot],
                              sem_g).wait()

        @pl.when(jnp.logical_and(lax.rem(j, GC) == GC - 1,
                                 lax.div(j, GC) < GN - 2))
        def _load_idx():
            g = lax.div(j, GC) + 2
            pltpu.async_copy(idx_h.at[cid, sid, g],
                             idx_v.at[pl.ds(lax.rem(g, 2) * GC, GC)], sem_i)

        @pl.when(j >= 1)
        def _drain_prev_out():
            pltpu.make_async_copy(
                buf.at[lax.rem(j + 2, 3)],
                out_h.at[cid].at[pl.ds(base + (j - 1) * GBLK, GBLK)],
                sem_o).wait()

        pltpu.async_copy(buf.at[slot],
                         out_h.at[cid].at[pl.ds(base + j * GBLK, GBLK)],
                         sem_o)

        @pl.when(j + 2 < TCW)
        def _prefetch():
            pltpu.async_copy(tab.at[idx_v.at[_irow(j + 2)]],
                             buf.at[lax.rem(j + 2, 3)], sem_g)
        return carry

    lax.fori_loop(0, TCW, chunk, 0)
    pltpu.make_async_copy(
        buf.at[lax.rem(TCW - 1, 3)],
        out_h.at[cid].at[pl.ds(base + (TCW - 1) * GBLK, GBLK)],
        sem_o).wait()


@functools.partial(
    pl.kernel,
    out_type=jax.ShapeDtypeStruct((NC, NACC, D), jnp.float32),
    mesh=plsc.VectorSubcoreMesh(**_MESH),
    scratch_types=[
        pltpu.VMEM((SCW, SBLK), jnp.int32),
        pltpu.VMEM((2, SBLK, D), jnp.float32),
        pltpu.VMEM((ZR, D), jnp.float32),
        pltpu.VMEM_SHARED((NACC, D), jnp.float32),
        pltpu.SemaphoreType.DMA,
        pltpu.SemaphoreType.DMA,
    ],
)
def _scatter_sc(ef2_h, idxs_h, out_h, idx_v, upd, zbuf, acc, sem, sem_i):
    cid = lax.axis_index("c")
    sid = lax.axis_index("s")
    wid = sid * NC + cid

    base_e = wid * SEW
    idx_cp = pltpu.async_copy(idxs_h.at[wid], idx_v, sem_i)
    pltpu.async_copy(ef2_h.at[pl.ds(base_e, SBLK)], upd.at[0], sem)

    def zrow(r, carry):
        for g in range(D // 16):
            zbuf[r, pl.ds(g * 16, 16)] = jnp.zeros((16,), jnp.float32)
        return carry

    lax.fori_loop(0, ZR, zrow, 0)

    base_r = sid * RPT
    for t in range(RPT // ZR):
        pltpu.sync_copy(zbuf, acc.at[pl.ds(base_r + t * ZR, ZR)])
    rem = RPT - (RPT // ZR) * ZR
    if rem:
        pltpu.sync_copy(zbuf.at[pl.ds(0, rem)],
                        acc.at[pl.ds(base_r + (RPT // ZR) * ZR, rem)])
    plsc.subcore_barrier()

    idx_cp.wait()

    def chunk(j, carry):
        slot = lax.rem(j, 2)
        nslot = lax.rem(j + 1, 2)

        @pl.when(j + 1 < SCW)
        def _prefetch():
            pltpu.async_copy(ef2_h.at[pl.ds(base_e + (j + 1) * SBLK, SBLK)],
                             upd.at[nslot], sem)

        pltpu.make_async_copy(ef2_h.at[pl.ds(base_e + j * SBLK, SBLK)],
                              upd.at[slot], sem).wait()
        pltpu.sync_copy(upd.at[slot], acc.at[idx_v.at[j]], add=True)
        return carry

    lax.fori_loop(0, SCW, chunk, 0)
    plsc.subcore_barrier()
    pltpu.sync_copy(acc.at[pl.ds(sid * RPT, RPT)],
                    out_h.at[cid].at[pl.ds(sid * RPT, RPT)])


def _silu(t):
    return t * jax.nn.sigmoid(t)


def _embed_body(h0_r, we_r, be_r, wa_r, wb_r, h_r, hp_r, hq_r):
    h = jnp.dot(h0_r[...], we_r[...],
                preferred_element_type=jnp.float32) + be_r[...]
    h_r[...] = h
    hp_r[...] = jnp.dot(h, wa_r[...], preferred_element_type=jnp.float32)
    hq_r[...] = jnp.dot(h, wb_r[...], preferred_element_type=jnp.float32)


def _edge_body(pre_r, ea8_r, weg_r, b1_r, w2_r, b2_r, out_r):
    f32 = jnp.float32
    t = pre_r[0] + pre_r[1] + jnp.dot(ea8_r[...], weg_r[...],
                                      preferred_element_type=f32) + b1_r[...]
    t = _silu(t)
    u = jnp.dot(t.astype(jnp.bfloat16), w2_r[...],
                preferred_element_type=f32) + b2_r[...]
    out_r[...] = _silu(u)


def _node_core(h, agg_r, h0_r, a_r, b_r, c_r, b1_r, w2_r, b2_r):
    f32 = jnp.float32
    agg = agg_r[0] + agg_r[1]
    t = (jnp.dot(h, a_r[...], preferred_element_type=f32)
         + jnp.dot(agg, b_r[...], preferred_element_type=f32)
         + jnp.dot(h0_r[...], c_r[...], preferred_element_type=f32)
         + b1_r[...])
    m = jnp.dot(_silu(t), w2_r[...], preferred_element_type=f32) + b2_r[...]
    return h + m


def _node_body(h_r, agg_r, h0_r, a_r, b_r, c_r, b1_r, w2_r, b2_r,
               wa_r, wb_r, hn_r, hp_r, hq_r):
    hn = _node_core(h_r[...], agg_r, h0_r, a_r, b_r, c_r, b1_r, w2_r, b2_r)
    hn_r[...] = hn
    hp_r[...] = jnp.dot(hn, wa_r[...], preferred_element_type=jnp.float32)
    hq_r[...] = jnp.dot(hn, wb_r[...], preferred_element_type=jnp.float32)


def _final_body(h_r, agg_r, h0_r, a_r, b_r, c_r, b1_r, w2_r, b2_r,
                ndw1_r, ndb1_r, ndw2_r, ndb2_r,
                gdw1_r, gdb1_r, gdw2_r, gdb2_r, out_r):
    f32 = jnp.float32
    hn = _node_core(h_r[...], agg_r, h0_r, a_r, b_r, c_r, b1_r, w2_r, b2_r)
    t = jnp.dot(_silu(jnp.dot(hn, ndw1_r[...], preferred_element_type=f32)
                      + ndb1_r[...]),
                ndw2_r[...], preferred_element_type=f32) + ndb2_r[...]
    u = _silu(jnp.dot(t, gdw1_r[...], preferred_element_type=f32)
              + gdb1_r[...])
    out_r[...] = jnp.dot(u, gdw2_r[...],
                         preferred_element_type=f32) + gdb2_r[...]


def _full(shape):
    return pl.BlockSpec(shape, lambda i: tuple(0 for _ in shape))


def _nblk():
    return pl.BlockSpec((BN, D), lambda i: (i, 0))


def _aggblk():
    return pl.BlockSpec((NC, BN, D), lambda i: (0, i, 0))


_W = _full((D, D))
_B = _full((1, D))


def _embed_call(h0, we, be, wa, wb):
    return pl.pallas_call(
        _embed_body,
        grid=(N // BN,),
        in_specs=[_nblk(), _W, _B, _W, _W],
        out_specs=[_nblk(), _nblk(), _nblk()],
        out_shape=[jax.ShapeDtypeStruct((N, D), jnp.float32)] * 3,
    )(h0, we, be, wa, wb)


def _edge_call(pre2, ea8, weg, b1, w2, b2):
    eblk = pl.BlockSpec((BE, D), lambda i: (i, 0))
    return pl.pallas_call(
        _edge_body,
        grid=(EPAD // BE,),
        in_specs=[pl.BlockSpec((NC, BE, D), lambda i: (0, i, 0)),
                  pl.BlockSpec((BE, 8), lambda i: (i, 0)),
                  _full((8, D)), _B, _W, _B],
        out_specs=eblk,
        out_shape=jax.ShapeDtypeStruct((EPAD, D), jnp.float32),
    )(pre2, ea8, weg, b1, w2.astype(jnp.bfloat16), b2)


def _node_call(h, aggp, h0, a, b, c, b1, w2, b2, wa, wb):
    return pl.pallas_call(
        _node_body,
        grid=(N // BN,),
        in_specs=[_nblk(), _aggblk(), _nblk(), _W, _W, _W, _B, _W, _B,
                  _W, _W],
        out_specs=[_nblk(), _nblk(), _nblk()],
        out_shape=[jax.ShapeDtypeStruct((N, D), jnp.float32)] * 3,
    )(h, aggp, h0, a, b, c, b1, w2, b2, wa, wb)


def _final_call(h, aggp, h0, a, b, c, b1, w2, b2, ndw1, ndb1, ndw2, ndb2,
                gdw1, gdb1, gdw2p, gdb2p):
    return pl.pallas_call(
        _final_body,
        grid=(N // BN,),
        in_specs=[_nblk(), _aggblk(), _nblk(), _W, _W, _W, _B, _W, _B,
                  _W, _B, _W, _B, _W, _B, _W, _B],
        out_specs=_nblk(),
        out_shape=jax.ShapeDtypeStruct((N, D), jnp.float32),
    )(h, aggp, h0, a, b, c, b1, w2, b2, ndw1, ndb1, ndw2, ndb2,
      gdw1, gdb1, gdw2p, gdb2p)


def kernel(h0, x, edges, edge_attr, emb_W, emb_b, e_W1, e_b1, e_W2, e_b2,
           n_W1, n_b1, n_W2, n_b2, nd_W1, nd_b1, nd_W2, nd_b2,
           gd_W1, gd_b1, gd_W2, gd_b2):
    f32 = jnp.float32
    row = edges[0]
    col = edges[1]
    pad = EPAD - E
    pidx = jnp.arange(pad, dtype=jnp.int32)
    row_gf = jnp.concatenate([row, pidx % N])
    col_gf = jnp.concatenate([col, (pidx * 7 + 3) % N])
    row_g = row_gf.reshape(NW, CW, BLK)
    col_g = col_gf.reshape(NW, CW, BLK)
    idx5 = jnp.stack([row_gf, col_gf]).reshape(2, NS, GN, GC, GBLK)
    row_s = jnp.concatenate([row, N + (pidx % SINK)]).reshape(NW, SCW, SBLK)

    ea_pad = jnp.pad(edge_attr, ((0, pad), (0, 0)))
    x0, x1, x2 = x[:, 0], x[:, 1], x[:, 2]

    wa = e_W1[:, :D, :]
    wb = e_W1[:, D:2 * D, :]
    wg = e_W1[:, 2 * D:, :]
    na = n_W1[:, :D, :]
    nb = n_W1[:, D:2 * D, :]
    nc_ = n_W1[:, 2 * D:, :]
    gdw2p = jnp.pad(gd_W2, ((0, 0), (0, D - OUT)))
    gdb2p = jnp.pad(gd_b2, (0, D - OUT))[None, :]

    h, hp, hq = _embed_call(h0, emb_W, emb_b[None, :], wa[0], wb[0])
    radial = _radial_sc(x0, x1, x2, row_g, col_g)
    ea8 = jnp.concatenate(
        [radial.reshape(EPAD, 1), ea_pad, jnp.zeros((EPAD, 3), f32)], axis=1)

    pred = None
    for i in range(L):
        weg = jnp.concatenate([wg[i], jnp.zeros((3, D), f32)], axis=0)
        pre2 = _gather_sc(hp, hq, idx5)
        ef2 = _edge_call(pre2, ea8, weg, e_b1[i][None, :], e_W2[i],
                         e_b2[i][None, :])
        aggp = _scatter_sc(ef2, row_s)
        if i < L - 1:
            h, hp, hq = _node_call(h, aggp, h0, na[i], nb[i], nc_[i],
                                   n_b1[i][None, :], n_W2[i],
                                   n_b2[i][None, :], wa[i + 1], wb[i + 1])
        else:
            pred = _final_call(h, aggp, h0, na[i], nb[i], nc_[i],
                               n_b1[i][None, :], n_W2[i], n_b2[i][None, :],
                               nd_W1, nd_b1[None, :], nd_W2, nd_b2[None, :],
                               gd_W1, gd_b1[None, :], gdw2p, gdb2p)
    return pred[:, :OUT]
